# Initial kernel scaffold; baseline (speedup 1.0000x reference)
import jax
import jax.numpy as jnp
from jax.experimental import pallas as pl


def kernel(atom_fea, nbr_fea, self_idx, nbr_idx, params):
    raise NotImplementedError("write your pallas kernel here")


# speedup vs baseline: 1.2766x; 1.2766x over previous
import functools

import jax
import jax.numpy as jnp
from jax import lax
from jax.experimental import pallas as pl
from jax.experimental.pallas import tpu as pltpu
from jax.experimental.pallas import tpu_sc as plsc

N = 10000
E = 320000
FEA = 128
NBR = 16
F2 = 256

NC = 2
NS = 16
NW = NC * NS
EPW = E // NW
CH = 80
NCHUNK = EPW // CH
ROWS_PER_SUB = 624
TAIL_ROW0 = NS * ROWS_PER_SUB
TAIL_ROWS = N - TAIL_ROW0


def _mmb_body(x_ref, wt_ref, b_ref, o_ref):
    o_ref[...] = (
        jnp.dot(x_ref[...], wt_ref[...], preferred_element_type=jnp.float32)
        + b_ref[...]
    )


def _mmb(x, wt, b):
    n, k = x.shape
    f = wt.shape[1]
    tn = 400
    return pl.pallas_call(
        _mmb_body,
        grid=(n // tn,),
        in_specs=[
            pl.BlockSpec((tn, k), lambda i: (i, 0)),
            pl.BlockSpec((k, f), lambda i: (0, 0)),
            pl.BlockSpec((1, f), lambda i: (0, 0)),
        ],
        out_specs=pl.BlockSpec((tn, f), lambda i: (i, 0)),
        out_shape=jax.ShapeDtypeStruct((n, f), jnp.float32),
    )(x, wt, b)


def _stats_body(gs_ref, gn_ref, nbr_ref, wft_ref, o_ref, acc):
    i = pl.program_id(0)
    y = gs_ref[...] + gn_ref[...] + jnp.dot(
        nbr_ref[...], wft_ref[...], preferred_element_type=jnp.float32)

    @pl.when(i == 0)
    def _():
        acc[...] = jnp.zeros_like(acc)

    acc[0:1, :] += jnp.sum(y, axis=0, keepdims=True)
    acc[1:2, :] += jnp.sum(y * y, axis=0, keepdims=True)

    @pl.when(i == pl.num_programs(0) - 1)
    def _():
        o_ref[...] = acc[...]


def _stats(gs, gn, nbr_fea, wft):
    te = 1000
    return pl.pallas_call(
        _stats_body,
        grid=(E // te,),
        in_specs=[
            pl.BlockSpec((te, F2), lambda i: (i, 0)),
            pl.BlockSpec((te, F2), lambda i: (i, 0)),
            pl.BlockSpec((te, NBR), lambda i: (i, 0)),
            pl.BlockSpec((NBR, F2), lambda i: (0, 0)),
        ],
        out_specs=pl.BlockSpec((2, F2), lambda i: (0, 0)),
        out_shape=jax.ShapeDtypeStruct((2, F2), jnp.float32),
        scratch_shapes=[pltpu.VMEM((2, F2), jnp.float32)],
    )(gs, gn, nbr_fea, wft)


def _softplus(u):
    return jnp.maximum(u, 0.0) + jnp.log1p(jnp.exp(-jnp.abs(u)))


def _msg_body(gs_ref, gn_ref, nbr_ref, wft_ref, st_ref, o_ref):
    y = gs_ref[...] + gn_ref[...] + jnp.dot(
        nbr_ref[...], wft_ref[...], preferred_element_type=jnp.float32)
    y = y * st_ref[0:1, :] + st_ref[1:2, :]
    filt = 1.0 / (1.0 + jnp.exp(-y[:, :FEA]))
    core = _softplus(y[:, FEA:])
    o_ref[...] = filt * core


def _msg(gs, gn, nbr_fea, wft, st):
    te = 1000
    return pl.pallas_call(
        _msg_body,
        grid=(E // te,),
        in_specs=[
            pl.BlockSpec((te, F2), lambda i: (i, 0)),
            pl.BlockSpec((te, F2), lambda i: (i, 0)),
            pl.BlockSpec((te, NBR), lambda i: (i, 0)),
            pl.BlockSpec((NBR, F2), lambda i: (0, 0)),
            pl.BlockSpec((2, F2), lambda i: (0, 0)),
        ],
        out_specs=pl.BlockSpec((te, FEA), lambda i: (i, 0)),
        out_shape=jax.ShapeDtypeStruct((E, FEA), jnp.float32),
    )(gs, gn, nbr_fea, wft, st)


def _final_body(x_ref, p_ref, g2_ref, b2_ref, o_ref):
    s = p_ref[0] + p_ref[1]
    mu = jnp.mean(s, axis=0, keepdims=True)
    var = jnp.mean(s * s, axis=0, keepdims=True) - mu * mu
    sn = (s - mu) / jnp.sqrt(var + 1e-5) * g2_ref[...] + b2_ref[...]
    o_ref[...] = _softplus(x_ref[...] + sn)


def _final(x, parts, g2, b2):
    return pl.pallas_call(
        _final_body,
        grid=(1,),
        in_specs=[
            pl.BlockSpec((N, FEA), lambda i: (0, 0)),
            pl.BlockSpec((2, N, FEA), lambda i: (0, 0, 0)),
            pl.BlockSpec((1, FEA), lambda i: (0, 0)),
            pl.BlockSpec((1, FEA), lambda i: (0, 0)),
        ],
        out_specs=pl.BlockSpec((N, FEA), lambda i: (0, 0)),
        out_shape=jax.ShapeDtypeStruct((N, FEA), jnp.float32),
    )(x, parts, g2, b2)


@functools.lru_cache(maxsize=None)
def _build_sc_gather():
    mesh = plsc.VectorSubcoreMesh(core_axis_name="c", subcore_axis_name="s")

    @functools.partial(
        pl.kernel,
        out_type=[
            jax.ShapeDtypeStruct((E, F2), jnp.float32),
            jax.ShapeDtypeStruct((E, F2), jnp.float32),
        ],
        mesh=mesh,
        scratch_types=[
            pltpu.VMEM((CH,), jnp.int32),
            pltpu.VMEM((CH,), jnp.int32),
            pltpu.VMEM((CH, F2), jnp.float32),
            pltpu.VMEM((CH, F2), jnp.float32),
            pltpu.SemaphoreType.DMA,
            pltpu.SemaphoreType.DMA,
        ],
    )
    def sc_gather(zs_hbm, zn_hbm, sidx_hbm, nidx_hbm, gs_hbm, gn_hbm,
                  idx_s, idx_n, buf_a, buf_b, sem_a, sem_b):
        wid = lax.axis_index("s") * NC + lax.axis_index("c")
        base = wid * EPW

        def body(k, carry):
            off = base + k * CH
            pltpu.sync_copy(sidx_hbm.at[pl.ds(off, CH)], idx_s)
            pltpu.sync_copy(nidx_hbm.at[pl.ds(off, CH)], idx_n)
            ca = pltpu.async_copy(zs_hbm.at[idx_s], buf_a, sem_a)
            cb = pltpu.async_copy(zn_hbm.at[idx_n], buf_b, sem_b)
            ca.wait()
            cb.wait()
            pltpu.sync_copy(buf_a, gs_hbm.at[pl.ds(off, CH)])
            pltpu.sync_copy(buf_b, gn_hbm.at[pl.ds(off, CH)])
            return carry

        lax.fori_loop(0, NCHUNK, body, 0)

    return sc_gather


def _sc_gather(zs, zn, sidx, nidx):
    return _build_sc_gather()(zs, zn, sidx, nidx)


@functools.lru_cache(maxsize=None)
def _build_sc_scatter():
    mesh = plsc.VectorSubcoreMesh(core_axis_name="c", subcore_axis_name="s")

    @functools.partial(
        pl.kernel,
        out_type=jax.ShapeDtypeStruct((NC, N, FEA), jnp.float32),
        mesh=mesh,
        scratch_types=[
            pltpu.VMEM((CH,), jnp.int32),
            pltpu.VMEM((CH, FEA), jnp.float32),
            pltpu.VMEM_SHARED((N, FEA), jnp.float32),
            pltpu.SemaphoreType.DMA,
        ],
    )
    def sc_scatter(msg_hbm, sidx_hbm, zeros_hbm, out_hbm,
                   idx_v, buf_v, acc_shared, sem):
        c = lax.axis_index("c")
        s = lax.axis_index("s")
        row0 = s * ROWS_PER_SUB
        pltpu.sync_copy(zeros_hbm.at[pl.ds(row0, ROWS_PER_SUB)],
                        acc_shared.at[pl.ds(row0, ROWS_PER_SUB)])

        @pl.when(s == NS - 1)
        def _():
            pltpu.sync_copy(zeros_hbm.at[pl.ds(TAIL_ROW0, TAIL_ROWS)],
                            acc_shared.at[pl.ds(TAIL_ROW0, TAIL_ROWS)])

        plsc.subcore_barrier()

        base = (s * NC + c) * EPW

        def body(k, carry):
            off = base + k * CH
            pltpu.sync_copy(sidx_hbm.at[pl.ds(off, CH)], idx_v)
            pltpu.sync_copy(msg_hbm.at[pl.ds(off, CH)], buf_v)
            pltpu.sync_copy(buf_v, acc_shared.at[idx_v], add=True)
            return carry

        lax.fori_loop(0, NCHUNK, body, 0)
        plsc.subcore_barrier()
        pltpu.sync_copy(acc_shared.at[pl.ds(row0, ROWS_PER_SUB)],
                        out_hbm.at[c].at[pl.ds(row0, ROWS_PER_SUB)])

        @pl.when(s == NS - 1)
        def _():
            pltpu.sync_copy(acc_shared.at[pl.ds(TAIL_ROW0, TAIL_ROWS)],
                            out_hbm.at[c].at[pl.ds(TAIL_ROW0, TAIL_ROWS)])

    return sc_scatter


def _sc_scatter(msg, sidx, zeros):
    return _build_sc_scatter()(msg, sidx, zeros)


def kernel(atom_fea, nbr_fea, self_idx, nbr_idx, params):
    p = params
    x = _mmb(atom_fea, p['emb_W'].T, p['emb_b'][None, :])
    zeros_nf = jnp.zeros((N, FEA), jnp.float32)
    zero_b = jnp.zeros((1, F2), jnp.float32)

    for c in p['convs']:
        wst = c['fc_W'][:, :FEA].T
        wnt = c['fc_W'][:, FEA:2 * FEA].T
        wft = c['fc_W'][:, 2 * FEA:].T

        zs = _mmb(x, wst, zero_b)
        zn = _mmb(x, wnt, c['fc_b'][None, :])
        gs, gn = _sc_gather(zs, zn, self_idx, nbr_idx)

        sums = _stats(gs, gn, nbr_fea, wft)
        mu = sums[0] / E
        var = sums[1] / E - mu * mu
        scale = c['bn1_g'] * lax.rsqrt(var + 1e-5)
        shift = c['bn1_b'] - mu * scale
        st = jnp.stack([scale, shift])

        msg = _msg(gs, gn, nbr_fea, wft, st)
        parts = _sc_scatter(msg, self_idx, zeros_nf)
        x = _final(x, parts, c['bn2_g'][None, :], c['bn2_b'][None, :])

    return x


# speedup vs baseline: 1.6471x; 1.2902x over previous
import functools

import jax
import jax.numpy as jnp
from jax import lax
from jax.experimental import pallas as pl
from jax.experimental.pallas import tpu as pltpu
from jax.experimental.pallas import tpu_sc as plsc

N = 10000
E = 320000
FEA = 128
NBR = 16
F2 = 256

NC = 2
NS = 16
NW = NC * NS
EPW = E // NW
CH = 80
NCHUNK = EPW // CH
ROWS_PER_SUB = 624
TAIL_ROW0 = NS * ROWS_PER_SUB
TAIL_ROWS = N - TAIL_ROW0


def _mmb_body(x_ref, wt_ref, b_ref, o_ref):
    o_ref[...] = (
        jnp.dot(x_ref[...], wt_ref[...], preferred_element_type=jnp.float32)
        + b_ref[...]
    )


def _mmb(x, wt, b):
    n, k = x.shape
    f = wt.shape[1]
    tn = 400
    return pl.pallas_call(
        _mmb_body,
        grid=(n // tn,),
        in_specs=[
            pl.BlockSpec((tn, k), lambda i: (i, 0)),
            pl.BlockSpec((k, f), lambda i: (0, 0)),
            pl.BlockSpec((1, f), lambda i: (0, 0)),
        ],
        out_specs=pl.BlockSpec((tn, f), lambda i: (i, 0)),
        out_shape=jax.ShapeDtypeStruct((n, f), jnp.float32),
    )(x, wt, b)


def _stats_body(g_ref, nbr_ref, wft_ref, o_ref, acc):
    i = pl.program_id(0)
    y = g_ref[...] + jnp.dot(
        nbr_ref[...], wft_ref[...], preferred_element_type=jnp.float32)

    @pl.when(i == 0)
    def _():
        acc[...] = jnp.zeros_like(acc)

    acc[0:1, :] += jnp.sum(y, axis=0, keepdims=True)
    acc[1:2, :] += jnp.sum(y * y, axis=0, keepdims=True)

    @pl.when(i == pl.num_programs(0) - 1)
    def _():
        o_ref[...] = acc[...]


def _stats(g, nbr_fea, wft):
    te = 1000
    return pl.pallas_call(
        _stats_body,
        grid=(E // te,),
        in_specs=[
            pl.BlockSpec((te, F2), lambda i: (i, 0)),
            pl.BlockSpec((te, NBR), lambda i: (i, 0)),
            pl.BlockSpec((NBR, F2), lambda i: (0, 0)),
        ],
        out_specs=pl.BlockSpec((2, F2), lambda i: (0, 0)),
        out_shape=jax.ShapeDtypeStruct((2, F2), jnp.float32),
        scratch_shapes=[pltpu.VMEM((2, F2), jnp.float32)],
    )(g, nbr_fea, wft)


def _softplus(u):
    return jnp.maximum(u, 0.0) + jnp.log1p(jnp.exp(-jnp.abs(u)))


def _msg_body(g_ref, nbr_ref, wft_ref, st_ref, o_ref):
    y = g_ref[...] + jnp.dot(
        nbr_ref[...], wft_ref[...], preferred_element_type=jnp.float32)
    y = y * st_ref[0:1, :] + st_ref[1:2, :]
    filt = 1.0 / (1.0 + jnp.exp(-y[:, :FEA]))
    core = _softplus(y[:, FEA:])
    o_ref[...] = filt * core


def _msg(g, nbr_fea, wft, st):
    te = 1000
    return pl.pallas_call(
        _msg_body,
        grid=(E // te,),
        in_specs=[
            pl.BlockSpec((te, F2), lambda i: (i, 0)),
            pl.BlockSpec((te, NBR), lambda i: (i, 0)),
            pl.BlockSpec((NBR, F2), lambda i: (0, 0)),
            pl.BlockSpec((2, F2), lambda i: (0, 0)),
        ],
        out_specs=pl.BlockSpec((te, FEA), lambda i: (i, 0)),
        out_shape=jax.ShapeDtypeStruct((E, FEA), jnp.float32),
    )(g, nbr_fea, wft, st)


def _final_body(x_ref, p_ref, g2_ref, b2_ref, o_ref):
    s = p_ref[0] + p_ref[1]
    mu = jnp.mean(s, axis=0, keepdims=True)
    var = jnp.mean(s * s, axis=0, keepdims=True) - mu * mu
    sn = (s - mu) / jnp.sqrt(var + 1e-5) * g2_ref[...] + b2_ref[...]
    o_ref[...] = _softplus(x_ref[...] + sn)


def _final(x, parts, g2, b2):
    return pl.pallas_call(
        _final_body,
        grid=(1,),
        in_specs=[
            pl.BlockSpec((N, FEA), lambda i: (0, 0)),
            pl.BlockSpec((2, N, FEA), lambda i: (0, 0, 0)),
            pl.BlockSpec((1, FEA), lambda i: (0, 0)),
            pl.BlockSpec((1, FEA), lambda i: (0, 0)),
        ],
        out_specs=pl.BlockSpec((N, FEA), lambda i: (0, 0)),
        out_shape=jax.ShapeDtypeStruct((N, FEA), jnp.float32),
    )(x, parts, g2, b2)


@functools.lru_cache(maxsize=None)
def _build_sc_gather():
    mesh = plsc.VectorSubcoreMesh(core_axis_name="c", subcore_axis_name="s")

    @functools.partial(
        pl.kernel,
        out_type=jax.ShapeDtypeStruct((E, F2), jnp.float32),
        mesh=mesh,
        scratch_types=[
            pltpu.VMEM((EPW,), jnp.int32),
            pltpu.VMEM((EPW,), jnp.int32),
            pltpu.VMEM((CH, F2), jnp.float32),
            pltpu.VMEM((CH, F2), jnp.float32),
            pltpu.VMEM((CH, F2), jnp.float32),
            pltpu.VMEM((CH, F2), jnp.float32),
            pltpu.SemaphoreType.DMA,
            pltpu.SemaphoreType.DMA,
            pltpu.SemaphoreType.DMA,
            pltpu.SemaphoreType.DMA,
        ],
    )
    def sc_gather(zs_hbm, zn_hbm, sidx_hbm, nidx_hbm, g_hbm,
                  idx_s, idx_n, a0, b0, a1, b1, sg0, sg1, sw0, sw1):
        wid = lax.axis_index("s") * NC + lax.axis_index("c")
        base = wid * EPW
        pltpu.sync_copy(sidx_hbm.at[pl.ds(base, EPW)], idx_s)
        pltpu.sync_copy(nidx_hbm.at[pl.ds(base, EPW)], idx_n)

        def fire_g(k, ba, bb, sem):
            ca = pltpu.async_copy(
                zs_hbm.at[idx_s.at[pl.ds(k * CH, CH)]], ba, sem)
            cb = pltpu.async_copy(
                zn_hbm.at[idx_n.at[pl.ds(k * CH, CH)]], bb, sem)
            return ca, cb

        def vadd(ba, bb):
            def row(r, carry):
                for v in range(F2 // 16):
                    sl = pl.ds(v * 16, 16)
                    ba[r, sl] = ba[r, sl] + bb[r, sl]
                return carry
            lax.fori_loop(0, CH, row, 0, unroll=2)

        fire_g(0, a0, b0, sg0)

        def body(j, carry):
            k0 = 2 * j
            k1 = k0 + 1
            c1a, c1b = fire_g(k1, a1, b1, sg1)
            w0a = pltpu.make_async_copy(zs_hbm.at[idx_s.at[pl.ds(0, CH)]],
                                        a0, sg0)
            w0b = pltpu.make_async_copy(zn_hbm.at[idx_n.at[pl.ds(0, CH)]],
                                        b0, sg0)
            w0a.wait()
            w0b.wait()
            vadd(a0, b0)
            wb0 = pltpu.async_copy(a0, g_hbm.at[pl.ds(base + k0 * CH, CH)],
                                   sw0)
            c1a.wait()
            c1b.wait()
            vadd(a1, b1)
            wb1 = pltpu.async_copy(a1, g_hbm.at[pl.ds(base + k1 * CH, CH)],
                                   sw1)
            wb0.wait()

            @pl.when(k0 + 2 < NCHUNK - 1)
            def _():
                fire_g(k0 + 2, a0, b0, sg0)

            wb1.wait()
            return carry

        lax.fori_loop(0, NCHUNK // 2, body, 0)

        kt = NCHUNK - 1
        ta, tb = fire_g(kt, a0, b0, sg0)
        ta.wait()
        tb.wait()
        vadd(a0, b0)
        pltpu.sync_copy(a0, g_hbm.at[pl.ds(base + kt * CH, CH)])

    return sc_gather


def _sc_gather(zs, zn, sidx, nidx):
    return _build_sc_gather()(zs, zn, sidx, nidx)


@functools.lru_cache(maxsize=None)
def _build_sc_scatter():
    mesh = plsc.VectorSubcoreMesh(core_axis_name="c", subcore_axis_name="s")

    @functools.partial(
        pl.kernel,
        out_type=jax.ShapeDtypeStruct((NC, N, FEA), jnp.float32),
        mesh=mesh,
        scratch_types=[
            pltpu.VMEM((CH,), jnp.int32),
            pltpu.VMEM((CH, FEA), jnp.float32),
            pltpu.VMEM_SHARED((N, FEA), jnp.float32),
            pltpu.SemaphoreType.DMA,
        ],
    )
    def sc_scatter(msg_hbm, sidx_hbm, zeros_hbm, out_hbm,
                   idx_v, buf_v, acc_shared, sem):
        c = lax.axis_index("c")
        s = lax.axis_index("s")
        row0 = s * ROWS_PER_SUB
        pltpu.sync_copy(zeros_hbm.at[pl.ds(row0, ROWS_PER_SUB)],
                        acc_shared.at[pl.ds(row0, ROWS_PER_SUB)])

        @pl.when(s == NS - 1)
        def _():
            pltpu.sync_copy(zeros_hbm.at[pl.ds(TAIL_ROW0, TAIL_ROWS)],
                            acc_shared.at[pl.ds(TAIL_ROW0, TAIL_ROWS)])

        plsc.subcore_barrier()

        base = (s * NC + c) * EPW

        def body(k, carry):
            off = base + k * CH
            pltpu.sync_copy(sidx_hbm.at[pl.ds(off, CH)], idx_v)
            pltpu.sync_copy(msg_hbm.at[pl.ds(off, CH)], buf_v)
            pltpu.sync_copy(buf_v, acc_shared.at[idx_v], add=True)
            return carry

        lax.fori_loop(0, NCHUNK, body, 0)
        plsc.subcore_barrier()
        pltpu.sync_copy(acc_shared.at[pl.ds(row0, ROWS_PER_SUB)],
                        out_hbm.at[c].at[pl.ds(row0, ROWS_PER_SUB)])

        @pl.when(s == NS - 1)
        def _():
            pltpu.sync_copy(acc_shared.at[pl.ds(TAIL_ROW0, TAIL_ROWS)],
                            out_hbm.at[c].at[pl.ds(TAIL_ROW0, TAIL_ROWS)])

    return sc_scatter


def _sc_scatter(msg, sidx, zeros):
    return _build_sc_scatter()(msg, sidx, zeros)


def kernel(atom_fea, nbr_fea, self_idx, nbr_idx, params):
    p = params
    x = _mmb(atom_fea, p['emb_W'].T, p['emb_b'][None, :])
    zeros_nf = jnp.zeros((N, FEA), jnp.float32)
    zero_b = jnp.zeros((1, F2), jnp.float32)

    for c in p['convs']:
        wst = c['fc_W'][:, :FEA].T
        wnt = c['fc_W'][:, FEA:2 * FEA].T
        wft = c['fc_W'][:, 2 * FEA:].T

        zs = _mmb(x, wst, zero_b)
        zn = _mmb(x, wnt, c['fc_b'][None, :])
        g = _sc_gather(zs, zn, self_idx, nbr_idx)

        sums = _stats(g, nbr_fea, wft)
        mu = sums[0] / E
        var = sums[1] / E - mu * mu
        scale = c['bn1_g'] * lax.rsqrt(var + 1e-5)
        shift = c['bn1_b'] - mu * scale
        st = jnp.stack([scale, shift])

        msg = _msg(g, nbr_fea, wft, st)
        parts = _sc_scatter(msg, self_idx, zeros_nf)
        x = _final(x, parts, c['bn2_g'][None, :], c['bn2_b'][None, :])

    return x


# speedup vs baseline: 1.7695x; 1.0743x over previous
import functools

import jax
import jax.numpy as jnp
from jax import lax
from jax.experimental import pallas as pl
from jax.experimental.pallas import tpu as pltpu
from jax.experimental.pallas import tpu_sc as plsc

N = 10000
E = 320000
FEA = 128
NBR = 16
F2 = 256

NC = 2
NS = 16
NW = NC * NS
EPW = E // NW
CH = 80
NCHUNK = EPW // CH
ROWS_PER_SUB = 624
TAIL_ROW0 = NS * ROWS_PER_SUB
TAIL_ROWS = N - TAIL_ROW0


def _embed_z_body(a_ref, ewt_ref, eb_ref, wst_ref, wnt_ref, fb_ref,
                  x_ref, zs_ref, zn_ref):
    x = jnp.dot(a_ref[...], ewt_ref[...],
                preferred_element_type=jnp.float32) + eb_ref[...]
    x_ref[...] = x
    zs_ref[...] = jnp.dot(x, wst_ref[...], preferred_element_type=jnp.float32)
    zn_ref[...] = jnp.dot(x, wnt_ref[...],
                          preferred_element_type=jnp.float32) + fb_ref[...]


def _embed_z(atom_fea, ewt, eb, wst, wnt, fb):
    full = lambda *s: pl.BlockSpec(s, lambda: tuple(0 for _ in s))
    return pl.pallas_call(
        _embed_z_body,
        in_specs=[
            full(N, FEA), full(FEA, FEA), full(1, FEA),
            full(FEA, F2), full(FEA, F2), full(1, F2),
        ],
        out_specs=[full(N, FEA), full(N, F2), full(N, F2)],
        out_shape=[
            jax.ShapeDtypeStruct((N, FEA), jnp.float32),
            jax.ShapeDtypeStruct((N, F2), jnp.float32),
            jax.ShapeDtypeStruct((N, F2), jnp.float32),
        ],
    )(atom_fea, ewt, eb, wst, wnt, fb)


def _bn2_res(x, parts, g2, b2):
    s = parts[0] + parts[1]
    mu = jnp.mean(s, axis=0, keepdims=True)
    var = jnp.mean(s * s, axis=0, keepdims=True) - mu * mu
    sn = (s - mu) / jnp.sqrt(var + 1e-5) * g2 + b2
    return _softplus(x + sn)


def _final_z_body(x_ref, p_ref, g2_ref, b2_ref, wst_ref, wnt_ref, fb_ref,
                  x_out, zs_ref, zn_ref):
    x = _bn2_res(x_ref[...], p_ref, g2_ref[...], b2_ref[...])
    x_out[...] = x
    zs_ref[...] = jnp.dot(x, wst_ref[...], preferred_element_type=jnp.float32)
    zn_ref[...] = jnp.dot(x, wnt_ref[...],
                          preferred_element_type=jnp.float32) + fb_ref[...]


def _final_z(x, parts, g2, b2, wst, wnt, fb):
    full = lambda *s: pl.BlockSpec(s, lambda: tuple(0 for _ in s))
    return pl.pallas_call(
        _final_z_body,
        in_specs=[
            full(N, FEA), full(NC, N, FEA), full(1, FEA), full(1, FEA),
            full(FEA, F2), full(FEA, F2), full(1, F2),
        ],
        out_specs=[full(N, FEA), full(N, F2), full(N, F2)],
        out_shape=[
            jax.ShapeDtypeStruct((N, FEA), jnp.float32),
            jax.ShapeDtypeStruct((N, F2), jnp.float32),
            jax.ShapeDtypeStruct((N, F2), jnp.float32),
        ],
    )(x, parts, g2, b2, wst, wnt, fb)


def _stats_body(g_ref, nbr_ref, wft_ref, o_ref, acc):
    i = pl.program_id(0)
    y = g_ref[...] + jnp.dot(
        nbr_ref[...], wft_ref[...], preferred_element_type=jnp.float32)

    @pl.when(i == 0)
    def _():
        acc[...] = jnp.zeros_like(acc)

    acc[0:1, :] += jnp.sum(y, axis=0, keepdims=True)
    acc[1:2, :] += jnp.sum(y * y, axis=0, keepdims=True)

    @pl.when(i == pl.num_programs(0) - 1)
    def _():
        o_ref[...] = acc[...]


def _stats(g, nbr_fea, wft):
    te = 1000
    return pl.pallas_call(
        _stats_body,
        grid=(E // te,),
        in_specs=[
            pl.BlockSpec((te, F2), lambda i: (i, 0)),
            pl.BlockSpec((te, NBR), lambda i: (i, 0)),
            pl.BlockSpec((NBR, F2), lambda i: (0, 0)),
        ],
        out_specs=pl.BlockSpec((2, F2), lambda i: (0, 0)),
        out_shape=jax.ShapeDtypeStruct((2, F2), jnp.float32),
        scratch_shapes=[pltpu.VMEM((2, F2), jnp.float32)],
    )(g, nbr_fea, wft)


def _softplus(u):
    return jnp.maximum(u, 0.0) + jnp.log1p(jnp.exp(-jnp.abs(u)))


def _msg_body(g_ref, nbr_ref, wft_ref, st_ref, o_ref):
    y = g_ref[...] + jnp.dot(
        nbr_ref[...], wft_ref[...], preferred_element_type=jnp.float32)
    y = y * st_ref[0:1, :] + st_ref[1:2, :]
    filt = 1.0 / (1.0 + jnp.exp(-y[:, :FEA]))
    core = _softplus(y[:, FEA:])
    o_ref[...] = filt * core


def _msg(g, nbr_fea, wft, st):
    te = 1000
    return pl.pallas_call(
        _msg_body,
        grid=(E // te,),
        in_specs=[
            pl.BlockSpec((te, F2), lambda i: (i, 0)),
            pl.BlockSpec((te, NBR), lambda i: (i, 0)),
            pl.BlockSpec((NBR, F2), lambda i: (0, 0)),
            pl.BlockSpec((2, F2), lambda i: (0, 0)),
        ],
        out_specs=pl.BlockSpec((te, FEA), lambda i: (i, 0)),
        out_shape=jax.ShapeDtypeStruct((E, FEA), jnp.float32),
    )(g, nbr_fea, wft, st)


def _final_body(x_ref, p_ref, g2_ref, b2_ref, o_ref):
    o_ref[...] = _bn2_res(x_ref[...], p_ref, g2_ref[...], b2_ref[...])


def _final(x, parts, g2, b2):
    full = lambda *s: pl.BlockSpec(s, lambda: tuple(0 for _ in s))
    return pl.pallas_call(
        _final_body,
        in_specs=[
            full(N, FEA), full(NC, N, FEA), full(1, FEA), full(1, FEA),
        ],
        out_specs=full(N, FEA),
        out_shape=jax.ShapeDtypeStruct((N, FEA), jnp.float32),
    )(x, parts, g2, b2)


@functools.lru_cache(maxsize=None)
def _build_sc_gather():
    mesh = plsc.VectorSubcoreMesh(core_axis_name="c", subcore_axis_name="s")

    @functools.partial(
        pl.kernel,
        out_type=jax.ShapeDtypeStruct((E, F2), jnp.float32),
        mesh=mesh,
        scratch_types=[
            pltpu.VMEM((NCHUNK, CH), jnp.int32),
            pltpu.VMEM((NCHUNK, CH), jnp.int32),
            pltpu.VMEM((CH, F2), jnp.float32),
            pltpu.VMEM((CH, F2), jnp.float32),
            pltpu.VMEM((CH, F2), jnp.float32),
            pltpu.VMEM((CH, F2), jnp.float32),
            pltpu.SemaphoreType.DMA,
            pltpu.SemaphoreType.DMA,
            pltpu.SemaphoreType.DMA,
            pltpu.SemaphoreType.DMA,
        ],
    )
    def sc_gather(zs_hbm, zn_hbm, sidx_hbm, nidx_hbm, g_hbm,
                  idx_s, idx_n, a0, b0, a1, b1, sg0, sg1, sw0, sw1):
        wid = lax.axis_index("s") * NC + lax.axis_index("c")
        base = wid * EPW
        pltpu.sync_copy(sidx_hbm.at[wid], idx_s)
        pltpu.sync_copy(nidx_hbm.at[wid], idx_n)

        def fire_g(k, ba, bb, sem):
            ca = pltpu.async_copy(zs_hbm.at[idx_s.at[k]], ba, sem)
            cb = pltpu.async_copy(zn_hbm.at[idx_n.at[k]], bb, sem)
            return ca, cb

        def vadd(ba, bb):
            def row(r, carry):
                for v in range(F2 // 16):
                    sl = pl.ds(v * 16, 16)
                    ba[r, sl] = ba[r, sl] + bb[r, sl]
                return carry
            lax.fori_loop(0, CH, row, 0, unroll=2)

        fire_g(0, a0, b0, sg0)

        def body(j, carry):
            k0 = 2 * j
            k1 = k0 + 1
            c1a, c1b = fire_g(k1, a1, b1, sg1)
            w0a = pltpu.make_async_copy(zs_hbm.at[idx_s.at[0]], a0, sg0)
            w0b = pltpu.make_async_copy(zn_hbm.at[idx_n.at[0]], b0, sg0)
            w0a.wait()
            w0b.wait()
            vadd(a0, b0)
            wb0 = pltpu.async_copy(a0, g_hbm.at[pl.ds(base + k0 * CH, CH)],
                                   sw0)
            c1a.wait()
            c1b.wait()
            vadd(a1, b1)
            wb1 = pltpu.async_copy(a1, g_hbm.at[pl.ds(base + k1 * CH, CH)],
                                   sw1)
            wb0.wait()

            @pl.when(k0 + 2 < NCHUNK - 1)
            def _():
                fire_g(k0 + 2, a0, b0, sg0)

            wb1.wait()
            return carry

        lax.fori_loop(0, NCHUNK // 2, body, 0)

        kt = NCHUNK - 1
        ta, tb = fire_g(kt, a0, b0, sg0)
        ta.wait()
        tb.wait()
        vadd(a0, b0)
        pltpu.sync_copy(a0, g_hbm.at[pl.ds(base + kt * CH, CH)])

    return sc_gather


def _sc_gather(zs, zn, sidx, nidx):
    return _build_sc_gather()(zs, zn, sidx, nidx)


@functools.lru_cache(maxsize=None)
def _build_sc_scatter():
    mesh = plsc.VectorSubcoreMesh(core_axis_name="c", subcore_axis_name="s")

    @functools.partial(
        pl.kernel,
        out_type=jax.ShapeDtypeStruct((NC, N, FEA), jnp.float32),
        mesh=mesh,
        scratch_types=[
            pltpu.VMEM((NCHUNK, CH), jnp.int32),
            pltpu.VMEM((CH, FEA), jnp.float32),
            pltpu.VMEM((CH, FEA), jnp.float32),
            pltpu.VMEM_SHARED((N, FEA), jnp.float32),
            pltpu.SemaphoreType.DMA,
            pltpu.SemaphoreType.DMA,
            pltpu.SemaphoreType.DMA,
            pltpu.SemaphoreType.DMA,
        ],
    )
    def sc_scatter(msg_hbm, sidx_hbm, zeros_hbm, out_hbm,
                   idx_v, m0, m1, acc_shared, sl0, sl1, sa0, sa1):
        c = lax.axis_index("c")
        s = lax.axis_index("s")
        row0 = s * ROWS_PER_SUB
        wid = s * NC + c
        base = wid * EPW
        pltpu.sync_copy(sidx_hbm.at[wid], idx_v)
        pltpu.sync_copy(zeros_hbm.at[pl.ds(row0, ROWS_PER_SUB)],
                        acc_shared.at[pl.ds(row0, ROWS_PER_SUB)])

        @pl.when(s == NS - 1)
        def _():
            pltpu.sync_copy(zeros_hbm.at[pl.ds(TAIL_ROW0, TAIL_ROWS)],
                            acc_shared.at[pl.ds(TAIL_ROW0, TAIL_ROWS)])

        plsc.subcore_barrier()

        def fire_load(k, buf, sem):
            return pltpu.async_copy(msg_hbm.at[pl.ds(base + k * CH, CH)],
                                    buf, sem)

        fire_load(0, m0, sl0)

        def body(j, carry):
            k0 = 2 * j
            k1 = k0 + 1
            l1 = fire_load(k1, m1, sl1)
            pltpu.make_async_copy(msg_hbm.at[pl.ds(base, CH)], m0, sl0).wait()
            a0 = pltpu.async_copy(m0, acc_shared.at[idx_v.at[k0]], sa0,
                                  add=True)
            l1.wait()
            a1 = pltpu.async_copy(m1, acc_shared.at[idx_v.at[k1]], sa1,
                                  add=True)
            a0.wait()

            @pl.when(k0 + 2 < NCHUNK - 1)
            def _():
                fire_load(k0 + 2, m0, sl0)

            a1.wait()
            return carry

        lax.fori_loop(0, NCHUNK // 2, body, 0)

        kt = NCHUNK - 1
        lt = fire_load(kt, m0, sl0)
        lt.wait()
        pltpu.sync_copy(m0, acc_shared.at[idx_v.at[kt]], add=True)

        plsc.subcore_barrier()
        pltpu.sync_copy(acc_shared.at[pl.ds(row0, ROWS_PER_SUB)],
                        out_hbm.at[c].at[pl.ds(row0, ROWS_PER_SUB)])

        @pl.when(s == NS - 1)
        def _():
            pltpu.sync_copy(acc_shared.at[pl.ds(TAIL_ROW0, TAIL_ROWS)],
                            out_hbm.at[c].at[pl.ds(TAIL_ROW0, TAIL_ROWS)])

    return sc_scatter


def _sc_scatter(msg, sidx, zeros):
    return _build_sc_scatter()(msg, sidx, zeros)


def kernel(atom_fea, nbr_fea, self_idx, nbr_idx, params):
    p = params
    convs = p['convs']
    sidx = self_idx.reshape(NW, NCHUNK, CH)
    nidx = nbr_idx.reshape(NW, NCHUNK, CH)
    zeros_nf = jnp.zeros((N, FEA), jnp.float32)

    def wsplit(c):
        return (c['fc_W'][:, :FEA].T, c['fc_W'][:, FEA:2 * FEA].T,
                c['fc_W'][:, 2 * FEA:].T, c['fc_b'][None, :])

    wst, wnt, wft, fb = wsplit(convs[0])
    x, zs, zn = _embed_z(atom_fea, p['emb_W'].T, p['emb_b'][None, :],
                         wst, wnt, fb)

    for li, c in enumerate(convs):
        g = _sc_gather(zs, zn, sidx, nidx)

        sums = _stats(g, nbr_fea, wft)
        mu = sums[0] / E
        var = sums[1] / E - mu * mu
        scale = c['bn1_g'] * lax.rsqrt(var + 1e-5)
        shift = c['bn1_b'] - mu * scale
        st = jnp.stack([scale, shift])

        msg = _msg(g, nbr_fea, wft, st)
        parts = _sc_scatter(msg, sidx, zeros_nf)
        g2 = c['bn2_g'][None, :]
        b2 = c['bn2_b'][None, :]
        if li + 1 < len(convs):
            wst, wnt, wft, fb = wsplit(convs[li + 1])
            x, zs, zn = _final_z(x, parts, g2, b2, wst, wnt, fb)
        else:
            x = _final(x, parts, g2, b2)

    return x


# speedup vs baseline: 2.0869x; 1.1793x over previous
import functools

import jax
import jax.numpy as jnp
from jax import lax
from jax.experimental import pallas as pl
from jax.experimental.pallas import tpu as pltpu
from jax.experimental.pallas import tpu_sc as plsc

N = 10000
E = 320000
FEA = 128
NBR = 16
F2 = 256

NC = 2
NS = 16
NW = NC * NS
EPW = E // NW
CH = 80
NCHUNK = EPW // CH
ROWS_PER_SUB = 624
TAIL_ROW0 = NS * ROWS_PER_SUB
TAIL_ROWS = N - TAIL_ROW0


def _pack16(v):
    b = jax.lax.bitcast_convert_type(v.astype(jnp.bfloat16), jnp.int16)
    return b.astype(jnp.int32) & 0xFFFF


def _store_z(z, z_ref):
    z_ref[...] = _pack16(z[:, :FEA]) | (_pack16(z[:, FEA:]) << 16)


def _unpack16(g):
    y0 = jax.lax.bitcast_convert_type(g << 16, jnp.float32)
    y1 = jax.lax.bitcast_convert_type(g & jnp.int32(-65536), jnp.float32)
    return y0, y1


def _embed_z_body(a_ref, ewt_ref, eb_ref, wst_ref, wnt_ref, fb_ref,
                  x_ref, zs_ref, zn_ref):
    x = jnp.dot(a_ref[...], ewt_ref[...],
                preferred_element_type=jnp.float32) + eb_ref[...]
    x_ref[...] = x
    _store_z(jnp.dot(x, wst_ref[...], preferred_element_type=jnp.float32),
             zs_ref)
    _store_z(jnp.dot(x, wnt_ref[...], preferred_element_type=jnp.float32)
             + fb_ref[...], zn_ref)


def _embed_z(atom_fea, ewt, eb, wst, wnt, fb):
    full = lambda *s: pl.BlockSpec(s, lambda: tuple(0 for _ in s))
    return pl.pallas_call(
        _embed_z_body,
        in_specs=[
            full(N, FEA), full(FEA, FEA), full(1, FEA),
            full(FEA, F2), full(FEA, F2), full(1, F2),
        ],
        out_specs=[full(N, FEA), full(N, FEA), full(N, FEA)],
        out_shape=[
            jax.ShapeDtypeStruct((N, FEA), jnp.float32),
            jax.ShapeDtypeStruct((N, FEA), jnp.int32),
            jax.ShapeDtypeStruct((N, FEA), jnp.int32),
        ],
    )(atom_fea, ewt, eb, wst, wnt, fb)


def _bn2_res(x, parts, g2, b2):
    s = parts[0] + parts[1]
    mu = jnp.mean(s, axis=0, keepdims=True)
    var = jnp.mean(s * s, axis=0, keepdims=True) - mu * mu
    sn = (s - mu) / jnp.sqrt(var + 1e-5) * g2 + b2
    return _softplus(x + sn)


def _final_z_body(x_ref, p_ref, g2_ref, b2_ref, wst_ref, wnt_ref, fb_ref,
                  x_out, zs_ref, zn_ref):
    x = _bn2_res(x_ref[...], p_ref, g2_ref[...], b2_ref[...])
    x_out[...] = x
    _store_z(jnp.dot(x, wst_ref[...], preferred_element_type=jnp.float32),
             zs_ref)
    _store_z(jnp.dot(x, wnt_ref[...], preferred_element_type=jnp.float32)
             + fb_ref[...], zn_ref)


def _final_z(x, parts, g2, b2, wst, wnt, fb):
    full = lambda *s: pl.BlockSpec(s, lambda: tuple(0 for _ in s))
    return pl.pallas_call(
        _final_z_body,
        in_specs=[
            full(N, FEA), full(NC, N, FEA), full(1, FEA), full(1, FEA),
            full(FEA, F2), full(FEA, F2), full(1, F2),
        ],
        out_specs=[full(N, FEA), full(N, FEA), full(N, FEA)],
        out_shape=[
            jax.ShapeDtypeStruct((N, FEA), jnp.float32),
            jax.ShapeDtypeStruct((N, FEA), jnp.int32),
            jax.ShapeDtypeStruct((N, FEA), jnp.int32),
        ],
    )(x, parts, g2, b2, wst, wnt, fb)


def _stats_body(gs_ref, gn_ref, nbr_ref, wft_ref, o_ref, acc):
    i = pl.program_id(0)
    c = jnp.dot(nbr_ref[...], wft_ref[...],
                preferred_element_type=jnp.float32)
    a0, a1 = _unpack16(gs_ref[...])
    b0, b1 = _unpack16(gn_ref[...])
    y0 = a0 + b0 + c[:, :FEA]
    y1 = a1 + b1 + c[:, FEA:]

    @pl.when(i == 0)
    def _():
        acc[...] = jnp.zeros_like(acc)

    acc[0:1, :] += jnp.sum(y0, axis=0, keepdims=True)
    acc[1:2, :] += jnp.sum(y1, axis=0, keepdims=True)
    acc[2:3, :] += jnp.sum(y0 * y0, axis=0, keepdims=True)
    acc[3:4, :] += jnp.sum(y1 * y1, axis=0, keepdims=True)

    @pl.when(i == pl.num_programs(0) - 1)
    def _():
        o_ref[...] = acc[...]


def _stats(gs, gn, nbr_fea, wft):
    te = 1000
    return pl.pallas_call(
        _stats_body,
        grid=(E // te,),
        in_specs=[
            pl.BlockSpec((te, FEA), lambda i: (i, 0)),
            pl.BlockSpec((te, FEA), lambda i: (i, 0)),
            pl.BlockSpec((te, NBR), lambda i: (i, 0)),
            pl.BlockSpec((NBR, F2), lambda i: (0, 0)),
        ],
        out_specs=pl.BlockSpec((4, FEA), lambda i: (0, 0)),
        out_shape=jax.ShapeDtypeStruct((4, FEA), jnp.float32),
        scratch_shapes=[pltpu.VMEM((4, FEA), jnp.float32)],
    )(gs, gn, nbr_fea, wft)


def _softplus(u):
    return jnp.maximum(u, 0.0) + jnp.log1p(jnp.exp(-jnp.abs(u)))


def _msg_body(gs_ref, gn_ref, nbr_ref, wft_ref, st_ref, o_ref):
    c = jnp.dot(nbr_ref[...], wft_ref[...],
                preferred_element_type=jnp.float32)
    a0, a1 = _unpack16(gs_ref[...])
    b0, b1 = _unpack16(gn_ref[...])
    y0 = a0 + b0 + c[:, :FEA]
    y1 = a1 + b1 + c[:, FEA:]
    y0 = y0 * st_ref[0:1, :] + st_ref[2:3, :]
    y1 = y1 * st_ref[1:2, :] + st_ref[3:4, :]
    filt = 1.0 / (1.0 + jnp.exp(-y0))
    core = _softplus(y1)
    o_ref[...] = filt * core


def _msg(gs, gn, nbr_fea, wft, st):
    te = 1000
    return pl.pallas_call(
        _msg_body,
        grid=(E // te,),
        in_specs=[
            pl.BlockSpec((te, FEA), lambda i: (i, 0)),
            pl.BlockSpec((te, FEA), lambda i: (i, 0)),
            pl.BlockSpec((te, NBR), lambda i: (i, 0)),
            pl.BlockSpec((NBR, F2), lambda i: (0, 0)),
            pl.BlockSpec((4, FEA), lambda i: (0, 0)),
        ],
        out_specs=pl.BlockSpec((te, FEA), lambda i: (i, 0)),
        out_shape=jax.ShapeDtypeStruct((E, FEA), jnp.float32),
    )(gs, gn, nbr_fea, wft, st)


def _final_body(x_ref, p_ref, g2_ref, b2_ref, o_ref):
    o_ref[...] = _bn2_res(x_ref[...], p_ref, g2_ref[...], b2_ref[...])


def _final(x, parts, g2, b2):
    full = lambda *s: pl.BlockSpec(s, lambda: tuple(0 for _ in s))
    return pl.pallas_call(
        _final_body,
        in_specs=[
            full(N, FEA), full(NC, N, FEA), full(1, FEA), full(1, FEA),
        ],
        out_specs=full(N, FEA),
        out_shape=jax.ShapeDtypeStruct((N, FEA), jnp.float32),
    )(x, parts, g2, b2)


@functools.lru_cache(maxsize=None)
def _build_sc_gather():
    mesh = plsc.VectorSubcoreMesh(core_axis_name="c", subcore_axis_name="s")

    @functools.partial(
        pl.kernel,
        out_type=[
            jax.ShapeDtypeStruct((E, FEA), jnp.int32),
            jax.ShapeDtypeStruct((E, FEA), jnp.int32),
        ],
        mesh=mesh,
        scratch_types=[
            pltpu.VMEM((NCHUNK, CH), jnp.int32),
            pltpu.VMEM((NCHUNK, CH), jnp.int32),
            pltpu.VMEM((CH, FEA), jnp.int32),
            pltpu.VMEM((CH, FEA), jnp.int32),
            pltpu.VMEM((CH, FEA), jnp.int32),
            pltpu.VMEM((CH, FEA), jnp.int32),
            pltpu.SemaphoreType.DMA,
            pltpu.SemaphoreType.DMA,
            pltpu.SemaphoreType.DMA,
            pltpu.SemaphoreType.DMA,
        ],
    )
    def sc_gather(zs_hbm, zn_hbm, sidx_hbm, nidx_hbm, gs_hbm, gn_hbm,
                  idx_s, idx_n, a0, b0, a1, b1, sg0, sg1, sw0, sw1):
        wid = lax.axis_index("s") * NC + lax.axis_index("c")
        base = wid * EPW
        pltpu.sync_copy(sidx_hbm.at[wid], idx_s)
        pltpu.sync_copy(nidx_hbm.at[wid], idx_n)

        def fire_g(k, ba, bb, sem):
            ca = pltpu.async_copy(zs_hbm.at[idx_s.at[k]], ba, sem)
            cb = pltpu.async_copy(zn_hbm.at[idx_n.at[k]], bb, sem)
            return ca, cb

        def wait_g(ba, bb, sem):
            pltpu.make_async_copy(zs_hbm.at[idx_s.at[0]], ba, sem).wait()
            pltpu.make_async_copy(zn_hbm.at[idx_n.at[0]], bb, sem).wait()

        def fire_w(k, ba, bb, sem):
            ca = pltpu.async_copy(ba, gs_hbm.at[pl.ds(base + k * CH, CH)],
                                  sem)
            cb = pltpu.async_copy(bb, gn_hbm.at[pl.ds(base + k * CH, CH)],
                                  sem)
            return ca, cb

        def wait_w(ba, bb, sem):
            pltpu.make_async_copy(ba, gs_hbm.at[pl.ds(base, CH)], sem).wait()
            pltpu.make_async_copy(bb, gn_hbm.at[pl.ds(base, CH)], sem).wait()

        fire_g(0, a0, b0, sg0)

        def body(j, carry):
            k0 = 2 * j
            k1 = k0 + 1
            fire_g(k1, a1, b1, sg1)
            wait_g(a0, b0, sg0)
            fire_w(k0, a0, b0, sw0)
            wait_g(a1, b1, sg1)
            fire_w(k1, a1, b1, sw1)
            wait_w(a0, b0, sw0)

            @pl.when(k0 + 2 < NCHUNK - 1)
            def _():
                fire_g(k0 + 2, a0, b0, sg0)

            wait_w(a1, b1, sw1)
            return carry

        lax.fori_loop(0, NCHUNK // 2, body, 0)

        kt = NCHUNK - 1
        fire_g(kt, a0, b0, sg0)
        wait_g(a0, b0, sg0)
        fire_w(kt, a0, b0, sw0)
        wait_w(a0, b0, sw0)

    return sc_gather


def _sc_gather(zs, zn, sidx, nidx):
    return _build_sc_gather()(zs, zn, sidx, nidx)


@functools.lru_cache(maxsize=None)
def _build_sc_scatter():
    mesh = plsc.VectorSubcoreMesh(core_axis_name="c", subcore_axis_name="s")

    @functools.partial(
        pl.kernel,
        out_type=jax.ShapeDtypeStruct((NC, N, FEA), jnp.float32),
        mesh=mesh,
        scratch_types=[
            pltpu.VMEM((NCHUNK, CH), jnp.int32),
            pltpu.VMEM((CH, FEA), jnp.float32),
            pltpu.VMEM((CH, FEA), jnp.float32),
            pltpu.VMEM_SHARED((N, FEA), jnp.float32),
            pltpu.SemaphoreType.DMA,
            pltpu.SemaphoreType.DMA,
            pltpu.SemaphoreType.DMA,
            pltpu.SemaphoreType.DMA,
        ],
    )
    def sc_scatter(msg_hbm, sidx_hbm, zeros_hbm, out_hbm,
                   idx_v, m0, m1, acc_shared, sl0, sl1, sa0, sa1):
        c = lax.axis_index("c")
        s = lax.axis_index("s")
        row0 = s * ROWS_PER_SUB
        wid = s * NC + c
        base = wid * EPW
        pltpu.sync_copy(sidx_hbm.at[wid], idx_v)
        pltpu.sync_copy(zeros_hbm.at[pl.ds(row0, ROWS_PER_SUB)],
                        acc_shared.at[pl.ds(row0, ROWS_PER_SUB)])

        @pl.when(s == NS - 1)
        def _():
            pltpu.sync_copy(zeros_hbm.at[pl.ds(TAIL_ROW0, TAIL_ROWS)],
                            acc_shared.at[pl.ds(TAIL_ROW0, TAIL_ROWS)])

        plsc.subcore_barrier()

        def fire_load(k, buf, sem):
            return pltpu.async_copy(msg_hbm.at[pl.ds(base + k * CH, CH)],
                                    buf, sem)

        fire_load(0, m0, sl0)

        def body(j, carry):
            k0 = 2 * j
            k1 = k0 + 1
            l1 = fire_load(k1, m1, sl1)
            pltpu.make_async_copy(msg_hbm.at[pl.ds(base, CH)], m0, sl0).wait()
            a0 = pltpu.async_copy(m0, acc_shared.at[idx_v.at[k0]], sa0,
                                  add=True)
            l1.wait()
            a1 = pltpu.async_copy(m1, acc_shared.at[idx_v.at[k1]], sa1,
                                  add=True)
            a0.wait()

            @pl.when(k0 + 2 < NCHUNK - 1)
            def _():
                fire_load(k0 + 2, m0, sl0)

            a1.wait()
            return carry

        lax.fori_loop(0, NCHUNK // 2, body, 0)

        kt = NCHUNK - 1
        lt = fire_load(kt, m0, sl0)
        lt.wait()
        pltpu.sync_copy(m0, acc_shared.at[idx_v.at[kt]], add=True)

        plsc.subcore_barrier()
        pltpu.sync_copy(acc_shared.at[pl.ds(row0, ROWS_PER_SUB)],
                        out_hbm.at[c].at[pl.ds(row0, ROWS_PER_SUB)])

        @pl.when(s == NS - 1)
        def _():
            pltpu.sync_copy(acc_shared.at[pl.ds(TAIL_ROW0, TAIL_ROWS)],
                            out_hbm.at[c].at[pl.ds(TAIL_ROW0, TAIL_ROWS)])

    return sc_scatter


def _sc_scatter(msg, sidx, zeros):
    return _build_sc_scatter()(msg, sidx, zeros)


def kernel(atom_fea, nbr_fea, self_idx, nbr_idx, params):
    p = params
    convs = p['convs']
    sidx = self_idx.reshape(NW, NCHUNK, CH)
    nidx = nbr_idx.reshape(NW, NCHUNK, CH)
    zeros_nf = jnp.zeros((N, FEA), jnp.float32)

    def wsplit(c):
        return (c['fc_W'][:, :FEA].T, c['fc_W'][:, FEA:2 * FEA].T,
                c['fc_W'][:, 2 * FEA:].T, c['fc_b'][None, :])

    wst, wnt, wft, fb = wsplit(convs[0])
    x, zs, zn = _embed_z(atom_fea, p['emb_W'].T, p['emb_b'][None, :],
                         wst, wnt, fb)

    for li, c in enumerate(convs):
        gs, gn = _sc_gather(zs, zn, sidx, nidx)

        sums = _stats(gs, gn, nbr_fea, wft)
        mu0, mu1 = sums[0] / E, sums[1] / E
        var0 = sums[2] / E - mu0 * mu0
        var1 = sums[3] / E - mu1 * mu1
        g1a, g1b = c['bn1_g'][:FEA], c['bn1_g'][FEA:]
        b1a, b1b = c['bn1_b'][:FEA], c['bn1_b'][FEA:]
        scale0 = g1a * lax.rsqrt(var0 + 1e-5)
        scale1 = g1b * lax.rsqrt(var1 + 1e-5)
        st = jnp.stack([scale0, scale1,
                        b1a - mu0 * scale0, b1b - mu1 * scale1])

        msg = _msg(gs, gn, nbr_fea, wft, st)
        parts = _sc_scatter(msg, sidx, zeros_nf)
        g2 = c['bn2_g'][None, :]
        b2 = c['bn2_b'][None, :]
        if li + 1 < len(convs):
            wst, wnt, wft, fb = wsplit(convs[li + 1])
            x, zs, zn = _final_z(x, parts, g2, b2, wst, wnt, fb)
        else:
            x = _final(x, parts, g2, b2)

    return x


# speedup vs baseline: 2.5796x; 1.2361x over previous
import functools

import jax
import jax.numpy as jnp
from jax import lax
from jax.experimental import pallas as pl
from jax.experimental.pallas import tpu as pltpu
from jax.experimental.pallas import tpu_sc as plsc

N = 10000
E = 320000
FEA = 128
NBR = 16
F2 = 256

NC = 2
NS = 16
NW = NC * NS
EPW = E // NW
CH = 80
NCHUNK = EPW // CH
ROWS_PER_SUB = 624
TAIL_ROW0 = NS * ROWS_PER_SUB
TAIL_ROWS = N - TAIL_ROW0


def _pack16(v):
    b = jax.lax.bitcast_convert_type(v.astype(jnp.bfloat16), jnp.int16)
    return b.astype(jnp.int32) & 0xFFFF


def _store_z(z, z_ref):
    z_ref[...] = _pack16(z[:, :FEA]) | (_pack16(z[:, FEA:]) << 16)


def _unpack16(g):
    y0 = jax.lax.bitcast_convert_type(g << 16, jnp.float32)
    y1 = jax.lax.bitcast_convert_type(g & jnp.int32(-65536), jnp.float32)
    return y0, y1


def _embed_z_body(a_ref, ewt_ref, eb_ref, wst_ref, wnt_ref, fb_ref,
                  x_ref, zs_ref, zn_ref):
    x = jnp.dot(a_ref[...], ewt_ref[...],
                preferred_element_type=jnp.float32) + eb_ref[...]
    x_ref[...] = x
    _store_z(jnp.dot(x, wst_ref[...], preferred_element_type=jnp.float32),
             zs_ref)
    _store_z(jnp.dot(x, wnt_ref[...], preferred_element_type=jnp.float32)
             + fb_ref[...], zn_ref)


def _embed_z(atom_fea, ewt, eb, wst, wnt, fb):
    full = lambda *s: pl.BlockSpec(s, lambda: tuple(0 for _ in s))
    return pl.pallas_call(
        _embed_z_body,
        in_specs=[
            full(N, FEA), full(FEA, FEA), full(1, FEA),
            full(FEA, F2), full(FEA, F2), full(1, F2),
        ],
        out_specs=[full(N, FEA), full(N, FEA), full(N, FEA)],
        out_shape=[
            jax.ShapeDtypeStruct((N, FEA), jnp.float32),
            jax.ShapeDtypeStruct((N, FEA), jnp.int32),
            jax.ShapeDtypeStruct((N, FEA), jnp.int32),
        ],
    )(atom_fea, ewt, eb, wst, wnt, fb)


def _bn2_res(x, parts, g2, b2):
    s = parts[0] + parts[1]
    mu = jnp.mean(s, axis=0, keepdims=True)
    var = jnp.mean(s * s, axis=0, keepdims=True) - mu * mu
    sn = (s - mu) / jnp.sqrt(var + 1e-5) * g2 + b2
    return _softplus(x + sn)


def _final_z_body(x_ref, p_ref, g2_ref, b2_ref, wst_ref, wnt_ref, fb_ref,
                  x_out, zs_ref, zn_ref):
    x = _bn2_res(x_ref[...], p_ref, g2_ref[...], b2_ref[...])
    x_out[...] = x
    _store_z(jnp.dot(x, wst_ref[...], preferred_element_type=jnp.float32),
             zs_ref)
    _store_z(jnp.dot(x, wnt_ref[...], preferred_element_type=jnp.float32)
             + fb_ref[...], zn_ref)


def _final_z(x, parts, g2, b2, wst, wnt, fb):
    full = lambda *s: pl.BlockSpec(s, lambda: tuple(0 for _ in s))
    return pl.pallas_call(
        _final_z_body,
        in_specs=[
            full(N, FEA), full(NC, N, FEA), full(1, FEA), full(1, FEA),
            full(FEA, F2), full(FEA, F2), full(1, F2),
        ],
        out_specs=[full(N, FEA), full(N, FEA), full(N, FEA)],
        out_shape=[
            jax.ShapeDtypeStruct((N, FEA), jnp.float32),
            jax.ShapeDtypeStruct((N, FEA), jnp.int32),
            jax.ShapeDtypeStruct((N, FEA), jnp.int32),
        ],
    )(x, parts, g2, b2, wst, wnt, fb)


def _stats_body(gs_ref, gn_ref, nbr_ref, wft_ref, o_ref, acc):
    i = pl.program_id(0)
    c = jnp.dot(nbr_ref[...], wft_ref[...],
                preferred_element_type=jnp.float32)
    a0, a1 = _unpack16(gs_ref[...])
    b0, b1 = _unpack16(gn_ref[...])
    y0 = a0 + b0 + c[:, :FEA]
    y1 = a1 + b1 + c[:, FEA:]

    @pl.when(i == 0)
    def _():
        acc[...] = jnp.zeros_like(acc)

    acc[0:1, :] += jnp.sum(y0, axis=0, keepdims=True)
    acc[1:2, :] += jnp.sum(y1, axis=0, keepdims=True)
    acc[2:3, :] += jnp.sum(y0 * y0, axis=0, keepdims=True)
    acc[3:4, :] += jnp.sum(y1 * y1, axis=0, keepdims=True)

    @pl.when(i == pl.num_programs(0) - 1)
    def _():
        o_ref[...] = acc[...]


def _stats(gs, gn, nbr_fea, wft):
    te = 2000
    return pl.pallas_call(
        _stats_body,
        grid=(E // te,),
        in_specs=[
            pl.BlockSpec((te, FEA), lambda i: (i, 0)),
            pl.BlockSpec((te, FEA), lambda i: (i, 0)),
            pl.BlockSpec((te, NBR), lambda i: (i, 0)),
            pl.BlockSpec((NBR, F2), lambda i: (0, 0)),
        ],
        out_specs=pl.BlockSpec((4, FEA), lambda i: (0, 0)),
        out_shape=jax.ShapeDtypeStruct((4, FEA), jnp.float32),
        scratch_shapes=[pltpu.VMEM((4, FEA), jnp.float32)],
    )(gs, gn, nbr_fea, wft)


def _softplus(u):
    return jnp.maximum(u, 0.0) + jnp.log1p(jnp.exp(-jnp.abs(u)))


def _msg_body(gs_ref, gn_ref, nbr_ref, wft_ref, sums_ref, bg_ref, bb_ref,
              o_ref):
    c = jnp.dot(nbr_ref[...], wft_ref[...],
                preferred_element_type=jnp.float32)
    mu0 = sums_ref[0:1, :] * (1.0 / E)
    mu1 = sums_ref[1:2, :] * (1.0 / E)
    var0 = sums_ref[2:3, :] * (1.0 / E) - mu0 * mu0
    var1 = sums_ref[3:4, :] * (1.0 / E) - mu1 * mu1
    sc0 = bg_ref[0:1, :] * lax.rsqrt(var0 + 1e-5)
    sc1 = bg_ref[1:2, :] * lax.rsqrt(var1 + 1e-5)
    sh0 = bb_ref[0:1, :] - mu0 * sc0
    sh1 = bb_ref[1:2, :] - mu1 * sc1
    a0, a1 = _unpack16(gs_ref[...])
    b0, b1 = _unpack16(gn_ref[...])
    y0 = (a0 + b0 + c[:, :FEA]) * sc0 + sh0
    y1 = (a1 + b1 + c[:, FEA:]) * sc1 + sh1
    filt = 1.0 / (1.0 + jnp.exp(-y0))
    core = _softplus(y1)
    o_ref[...] = filt * core


def _msg(gs, gn, nbr_fea, wft, sums, bg, bb):
    te = 2000
    return pl.pallas_call(
        _msg_body,
        grid=(E // te,),
        in_specs=[
            pl.BlockSpec((te, FEA), lambda i: (i, 0)),
            pl.BlockSpec((te, FEA), lambda i: (i, 0)),
            pl.BlockSpec((te, NBR), lambda i: (i, 0)),
            pl.BlockSpec((NBR, F2), lambda i: (0, 0)),
            pl.BlockSpec((4, FEA), lambda i: (0, 0)),
            pl.BlockSpec((2, FEA), lambda i: (0, 0)),
            pl.BlockSpec((2, FEA), lambda i: (0, 0)),
        ],
        out_specs=pl.BlockSpec((te, FEA), lambda i: (i, 0)),
        out_shape=jax.ShapeDtypeStruct((E, FEA), jnp.float32),
    )(gs, gn, nbr_fea, wft, sums, bg, bb)


def _final_body(x_ref, p_ref, g2_ref, b2_ref, o_ref):
    o_ref[...] = _bn2_res(x_ref[...], p_ref, g2_ref[...], b2_ref[...])


def _final(x, parts, g2, b2):
    full = lambda *s: pl.BlockSpec(s, lambda: tuple(0 for _ in s))
    return pl.pallas_call(
        _final_body,
        in_specs=[
            full(N, FEA), full(NC, N, FEA), full(1, FEA), full(1, FEA),
        ],
        out_specs=full(N, FEA),
        out_shape=jax.ShapeDtypeStruct((N, FEA), jnp.float32),
    )(x, parts, g2, b2)


@functools.lru_cache(maxsize=None)
def _build_sc_gather():
    mesh = plsc.VectorSubcoreMesh(core_axis_name="c", subcore_axis_name="s")

    @functools.partial(
        pl.kernel,
        out_type=[
            jax.ShapeDtypeStruct((E, FEA), jnp.int32),
            jax.ShapeDtypeStruct((E, FEA), jnp.int32),
        ],
        mesh=mesh,
        scratch_types=[
            pltpu.VMEM((NCHUNK, CH), jnp.int32),
            pltpu.VMEM((NCHUNK, CH), jnp.int32),
        ] + [pltpu.VMEM((CH, FEA), jnp.int32)] * 8
          + [pltpu.SemaphoreType.DMA] * 8,
    )
    def sc_gather(zs_hbm, zn_hbm, sidx_hbm, nidx_hbm, gs_hbm, gn_hbm,
                  idx_s, idx_n, a0, b0, a1, b1, a2, b2, a3, b3,
                  sg0, sg1, sg2, sg3, sw0, sw1, sw2, sw3):
        wid = lax.axis_index("s") * NC + lax.axis_index("c")
        base = wid * EPW
        pltpu.sync_copy(sidx_hbm.at[wid], idx_s)
        pltpu.sync_copy(nidx_hbm.at[wid], idx_n)

        def fire_g(k, ba, bb, sem):
            ca = pltpu.async_copy(zs_hbm.at[idx_s.at[k]], ba, sem)
            cb = pltpu.async_copy(zn_hbm.at[idx_n.at[k]], bb, sem)
            return ca, cb

        def wait_g(ba, bb, sem):
            pltpu.make_async_copy(zs_hbm.at[idx_s.at[0]], ba, sem).wait()
            pltpu.make_async_copy(zn_hbm.at[idx_n.at[0]], bb, sem).wait()

        def fire_w(k, ba, bb, sem):
            ca = pltpu.async_copy(ba, gs_hbm.at[pl.ds(base + k * CH, CH)],
                                  sem)
            cb = pltpu.async_copy(bb, gn_hbm.at[pl.ds(base + k * CH, CH)],
                                  sem)
            return ca, cb

        def wait_w(ba, bb, sem):
            pltpu.make_async_copy(ba, gs_hbm.at[pl.ds(base, CH)], sem).wait()
            pltpu.make_async_copy(bb, gn_hbm.at[pl.ds(base, CH)], sem).wait()

        slots = [(a0, b0, sg0), (a1, b1, sg1), (a2, b2, sg2), (a3, b3, sg3)]
        wsems = [sw0, sw1, sw2, sw3]
        nmain = NCHUNK - 1

        for b in range(4):
            ba, bb, sem = slots[b]
            fire_g(b, ba, bb, sem)

        def body(j, carry):
            for b in range(4):
                k = 4 * j + b
                ba, bb, sem = slots[b]
                wait_g(ba, bb, sem)
                fire_w(k, ba, bb, wsems[b])
            for b in range(4):
                k2 = 4 * j + 4 + b
                ba, bb, sem = slots[b]

                @pl.when(k2 < nmain)
                def _(k2=k2, ba=ba, bb=bb, sem=sem, b=b):
                    wait_w(ba, bb, wsems[b])
                    fire_g(k2, ba, bb, sem)

            return carry

        lax.fori_loop(0, nmain // 4, body, 0)
        for b in range(4):
            ba, bb, _ = slots[b]
            wait_w(ba, bb, wsems[b])

        kt = NCHUNK - 1
        fire_g(kt, a0, b0, sg0)
        wait_g(a0, b0, sg0)
        fire_w(kt, a0, b0, sw0)
        wait_w(a0, b0, sw0)

    return sc_gather


def _sc_gather(zs, zn, sidx, nidx):
    return _build_sc_gather()(zs, zn, sidx, nidx)


@functools.lru_cache(maxsize=None)
def _build_sc_scatter():
    mesh = plsc.VectorSubcoreMesh(core_axis_name="c", subcore_axis_name="s")

    @functools.partial(
        pl.kernel,
        out_type=jax.ShapeDtypeStruct((NC, N, FEA), jnp.float32),
        mesh=mesh,
        scratch_types=[
            pltpu.VMEM((NCHUNK, CH), jnp.int32),
            pltpu.VMEM((CH, FEA), jnp.float32),
            pltpu.VMEM((CH, FEA), jnp.float32),
            pltpu.VMEM_SHARED((N, FEA), jnp.float32),
            pltpu.SemaphoreType.DMA,
            pltpu.SemaphoreType.DMA,
            pltpu.SemaphoreType.DMA,
            pltpu.SemaphoreType.DMA,
        ],
    )
    def sc_scatter(msg_hbm, sidx_hbm, zeros_hbm, out_hbm,
                   idx_v, m0, m1, acc_shared, sl0, sl1, sa0, sa1):
        c = lax.axis_index("c")
        s = lax.axis_index("s")
        row0 = s * ROWS_PER_SUB
        wid = s * NC + c
        base = wid * EPW
        pltpu.sync_copy(sidx_hbm.at[wid], idx_v)
        pltpu.sync_copy(zeros_hbm.at[pl.ds(row0, ROWS_PER_SUB)],
                        acc_shared.at[pl.ds(row0, ROWS_PER_SUB)])

        @pl.when(s == NS - 1)
        def _():
            pltpu.sync_copy(zeros_hbm.at[pl.ds(TAIL_ROW0, TAIL_ROWS)],
                            acc_shared.at[pl.ds(TAIL_ROW0, TAIL_ROWS)])

        plsc.subcore_barrier()

        def fire_load(k, buf, sem):
            return pltpu.async_copy(msg_hbm.at[pl.ds(base + k * CH, CH)],
                                    buf, sem)

        fire_load(0, m0, sl0)

        def body(j, carry):
            k0 = 2 * j
            k1 = k0 + 1
            l1 = fire_load(k1, m1, sl1)
            pltpu.make_async_copy(msg_hbm.at[pl.ds(base, CH)], m0, sl0).wait()
            a0 = pltpu.async_copy(m0, acc_shared.at[idx_v.at[k0]], sa0,
                                  add=True)
            l1.wait()
            a1 = pltpu.async_copy(m1, acc_shared.at[idx_v.at[k1]], sa1,
                                  add=True)
            a0.wait()

            @pl.when(k0 + 2 < NCHUNK - 1)
            def _():
                fire_load(k0 + 2, m0, sl0)

            a1.wait()
            return carry

        lax.fori_loop(0, NCHUNK // 2, body, 0)

        kt = NCHUNK - 1
        lt = fire_load(kt, m0, sl0)
        lt.wait()
        pltpu.sync_copy(m0, acc_shared.at[idx_v.at[kt]], add=True)

        plsc.subcore_barrier()
        pltpu.sync_copy(acc_shared.at[pl.ds(row0, ROWS_PER_SUB)],
                        out_hbm.at[c].at[pl.ds(row0, ROWS_PER_SUB)])

        @pl.when(s == NS - 1)
        def _():
            pltpu.sync_copy(acc_shared.at[pl.ds(TAIL_ROW0, TAIL_ROWS)],
                            out_hbm.at[c].at[pl.ds(TAIL_ROW0, TAIL_ROWS)])

    return sc_scatter


def _sc_scatter(msg, sidx, zeros):
    return _build_sc_scatter()(msg, sidx, zeros)


def kernel(atom_fea, nbr_fea, self_idx, nbr_idx, params):
    p = params
    convs = p['convs']
    sidx = self_idx.reshape(NW, NCHUNK, CH)
    nidx = nbr_idx.reshape(NW, NCHUNK, CH)
    zeros_nf = jnp.zeros((N, FEA), jnp.float32)

    def wsplit(c):
        return (c['fc_W'][:, :FEA].T, c['fc_W'][:, FEA:2 * FEA].T,
                c['fc_W'][:, 2 * FEA:].T, c['fc_b'][None, :])

    wst, wnt, wft, fb = wsplit(convs[0])
    x, zs, zn = _embed_z(atom_fea, p['emb_W'].T, p['emb_b'][None, :],
                         wst, wnt, fb)

    for li, c in enumerate(convs):
        gs, gn = _sc_gather(zs, zn, sidx, nidx)

        sums = _stats(gs, gn, nbr_fea, wft)
        bg1 = c['bn1_g'].reshape(2, FEA)
        bb1 = c['bn1_b'].reshape(2, FEA)

        msg = _msg(gs, gn, nbr_fea, wft, sums, bg1, bb1)
        parts = _sc_scatter(msg, sidx, zeros_nf)
        g2 = c['bn2_g'][None, :]
        b2 = c['bn2_b'][None, :]
        if li + 1 < len(convs):
            wst, wnt, wft, fb = wsplit(convs[li + 1])
            x, zs, zn = _final_z(x, parts, g2, b2, wst, wnt, fb)
        else:
            x = _final(x, parts, g2, b2)

    return x


# speedup vs baseline: 2.6381x; 1.0227x over previous
import functools

import jax
import jax.numpy as jnp
from jax import lax
from jax.experimental import pallas as pl
from jax.experimental.pallas import tpu as pltpu
from jax.experimental.pallas import tpu_sc as plsc

N = 10000
E = 320000
FEA = 128
NBR = 16
F2 = 256

NC = 2
NS = 16
NW = NC * NS
EPW = E // NW
CH = 80
NCHUNK = EPW // CH
ROWS_PER_SUB = 624
TAIL_ROW0 = NS * ROWS_PER_SUB
TAIL_ROWS = N - TAIL_ROW0


def _pack16(v):
    b = jax.lax.bitcast_convert_type(v.astype(jnp.bfloat16), jnp.int16)
    return b.astype(jnp.int32) & 0xFFFF


def _store_z(z, z_ref):
    z_ref[...] = _pack16(z[:, :FEA]) | (_pack16(z[:, FEA:]) << 16)


def _unpack16(g):
    y0 = jax.lax.bitcast_convert_type(g << 16, jnp.float32)
    y1 = jax.lax.bitcast_convert_type(g & jnp.int32(-65536), jnp.float32)
    return y0, y1


def _embed_z_body(a_ref, ewt_ref, eb_ref, wst_ref, wnt_ref, fb_ref,
                  x_ref, zs_ref, zn_ref):
    x = jnp.dot(a_ref[...], ewt_ref[...],
                preferred_element_type=jnp.float32) + eb_ref[...]
    x_ref[...] = x
    _store_z(jnp.dot(x, wst_ref[...], preferred_element_type=jnp.float32),
             zs_ref)
    _store_z(jnp.dot(x, wnt_ref[...], preferred_element_type=jnp.float32)
             + fb_ref[...], zn_ref)


def _embed_z(atom_fea, ewt, eb, wst, wnt, fb):
    full = lambda *s: pl.BlockSpec(s, lambda: tuple(0 for _ in s))
    return pl.pallas_call(
        _embed_z_body,
        in_specs=[
            full(N, FEA), full(FEA, FEA), full(1, FEA),
            full(FEA, F2), full(FEA, F2), full(1, F2),
        ],
        out_specs=[full(N, FEA), full(N, FEA), full(N, FEA)],
        out_shape=[
            jax.ShapeDtypeStruct((N, FEA), jnp.float32),
            jax.ShapeDtypeStruct((N, FEA), jnp.int32),
            jax.ShapeDtypeStruct((N, FEA), jnp.int32),
        ],
    )(atom_fea, ewt, eb, wst, wnt, fb)


def _bn2_res(x, parts, g2, b2):
    s = parts[0] + parts[1]
    mu = jnp.mean(s, axis=0, keepdims=True)
    var = jnp.mean(s * s, axis=0, keepdims=True) - mu * mu
    sn = (s - mu) / jnp.sqrt(var + 1e-5) * g2 + b2
    return _softplus(x + sn)


def _final_z_body(x_ref, p_ref, g2_ref, b2_ref, wst_ref, wnt_ref, fb_ref,
                  x_out, zs_ref, zn_ref):
    x = _bn2_res(x_ref[...], p_ref, g2_ref[...], b2_ref[...])
    x_out[...] = x
    _store_z(jnp.dot(x, wst_ref[...], preferred_element_type=jnp.float32),
             zs_ref)
    _store_z(jnp.dot(x, wnt_ref[...], preferred_element_type=jnp.float32)
             + fb_ref[...], zn_ref)


def _final_z(x, parts, g2, b2, wst, wnt, fb):
    full = lambda *s: pl.BlockSpec(s, lambda: tuple(0 for _ in s))
    return pl.pallas_call(
        _final_z_body,
        in_specs=[
            full(N, FEA), full(NC, N, FEA), full(1, FEA), full(1, FEA),
            full(FEA, F2), full(FEA, F2), full(1, F2),
        ],
        out_specs=[full(N, FEA), full(N, FEA), full(N, FEA)],
        out_shape=[
            jax.ShapeDtypeStruct((N, FEA), jnp.float32),
            jax.ShapeDtypeStruct((N, FEA), jnp.int32),
            jax.ShapeDtypeStruct((N, FEA), jnp.int32),
        ],
    )(x, parts, g2, b2, wst, wnt, fb)


def _softplus(u):
    return jnp.maximum(u, 0.0) + jnp.log1p(jnp.exp(-jnp.abs(u)))


def _statsmsg_body(gs_ref, gn_ref, nbr_ref, wft_ref, bg_ref, bb_ref,
                   o_ref, acc):
    p = pl.program_id(0)
    i = pl.program_id(1)
    c = jnp.dot(nbr_ref[...], wft_ref[...],
                preferred_element_type=jnp.float32)
    a0, a1 = _unpack16(gs_ref[...])
    b0, b1 = _unpack16(gn_ref[...])
    y0 = a0 + b0 + c[:, :FEA]
    y1 = a1 + b1 + c[:, FEA:]

    @pl.when((p == 0) & (i == 0))
    def _():
        acc[...] = jnp.zeros_like(acc)

    @pl.when(p == 0)
    def _():
        acc[0:1, :] += jnp.sum(y0, axis=0, keepdims=True)
        acc[1:2, :] += jnp.sum(y1, axis=0, keepdims=True)
        acc[2:3, :] += jnp.sum(y0 * y0, axis=0, keepdims=True)
        acc[3:4, :] += jnp.sum(y1 * y1, axis=0, keepdims=True)

    @pl.when((p == 0) & (i == pl.num_programs(1) - 1))
    def _():
        mu0 = acc[0:1, :] * (1.0 / E)
        mu1 = acc[1:2, :] * (1.0 / E)
        var0 = acc[2:3, :] * (1.0 / E) - mu0 * mu0
        var1 = acc[3:4, :] * (1.0 / E) - mu1 * mu1
        sc0 = bg_ref[0:1, :] * lax.rsqrt(var0 + 1e-5)
        sc1 = bg_ref[1:2, :] * lax.rsqrt(var1 + 1e-5)
        acc[0:1, :] = sc0
        acc[1:2, :] = sc1
        acc[2:3, :] = bb_ref[0:1, :] - mu0 * sc0
        acc[3:4, :] = bb_ref[1:2, :] - mu1 * sc1

    @pl.when(p == 1)
    def _():
        z0 = y0 * acc[0:1, :] + acc[2:3, :]
        z1 = y1 * acc[1:2, :] + acc[3:4, :]
        filt = 1.0 / (1.0 + jnp.exp(-z0))
        core = _softplus(z1)
        o_ref[...] = filt * core


def _statsmsg(gs, gn, nbr_fea, wft, bg, bb):
    te = 2000
    return pl.pallas_call(
        _statsmsg_body,
        grid=(2, E // te),
        in_specs=[
            pl.BlockSpec((te, FEA), lambda p, i: (i, 0)),
            pl.BlockSpec((te, FEA), lambda p, i: (i, 0)),
            pl.BlockSpec((te, NBR), lambda p, i: (i, 0)),
            pl.BlockSpec((NBR, F2), lambda p, i: (0, 0)),
            pl.BlockSpec((2, FEA), lambda p, i: (0, 0)),
            pl.BlockSpec((2, FEA), lambda p, i: (0, 0)),
        ],
        out_specs=pl.BlockSpec((te, FEA), lambda p, i: (p * i, 0)),
        out_shape=jax.ShapeDtypeStruct((E, FEA), jnp.float32),
        scratch_shapes=[pltpu.VMEM((4, FEA), jnp.float32)],
    )(gs, gn, nbr_fea, wft, bg, bb)


def _final_body(x_ref, p_ref, g2_ref, b2_ref, o_ref):
    o_ref[...] = _bn2_res(x_ref[...], p_ref, g2_ref[...], b2_ref[...])


def _final(x, parts, g2, b2):
    full = lambda *s: pl.BlockSpec(s, lambda: tuple(0 for _ in s))
    return pl.pallas_call(
        _final_body,
        in_specs=[
            full(N, FEA), full(NC, N, FEA), full(1, FEA), full(1, FEA),
        ],
        out_specs=full(N, FEA),
        out_shape=jax.ShapeDtypeStruct((N, FEA), jnp.float32),
    )(x, parts, g2, b2)


@functools.lru_cache(maxsize=None)
def _build_sc_gather():
    mesh = plsc.VectorSubcoreMesh(core_axis_name="c", subcore_axis_name="s")

    @functools.partial(
        pl.kernel,
        out_type=[
            jax.ShapeDtypeStruct((E, FEA), jnp.int32),
            jax.ShapeDtypeStruct((E, FEA), jnp.int32),
        ],
        mesh=mesh,
        scratch_types=[
            pltpu.VMEM((NCHUNK, CH), jnp.int32),
            pltpu.VMEM((NCHUNK, CH), jnp.int32),
        ] + [pltpu.VMEM((CH, FEA), jnp.int32)] * 8
          + [pltpu.SemaphoreType.DMA] * 8,
    )
    def sc_gather(zs_hbm, zn_hbm, sidx_hbm, nidx_hbm, gs_hbm, gn_hbm,
                  idx_s, idx_n, a0, b0, a1, b1, a2, b2, a3, b3,
                  sg0, sg1, sg2, sg3, sw0, sw1, sw2, sw3):
        wid = lax.axis_index("s") * NC + lax.axis_index("c")
        base = wid * EPW
        pltpu.sync_copy(sidx_hbm.at[wid], idx_s)
        pltpu.sync_copy(nidx_hbm.at[wid], idx_n)

        def fire_g(k, ba, bb, sem):
            ca = pltpu.async_copy(zs_hbm.at[idx_s.at[k]], ba, sem)
            cb = pltpu.async_copy(zn_hbm.at[idx_n.at[k]], bb, sem)
            return ca, cb

        def wait_g(ba, bb, sem):
            pltpu.make_async_copy(zs_hbm.at[idx_s.at[0]], ba, sem).wait()
            pltpu.make_async_copy(zn_hbm.at[idx_n.at[0]], bb, sem).wait()

        def fire_w(k, ba, bb, sem):
            ca = pltpu.async_copy(ba, gs_hbm.at[pl.ds(base + k * CH, CH)],
                                  sem)
            cb = pltpu.async_copy(bb, gn_hbm.at[pl.ds(base + k * CH, CH)],
                                  sem)
            return ca, cb

        def wait_w(ba, bb, sem):
            pltpu.make_async_copy(ba, gs_hbm.at[pl.ds(base, CH)], sem).wait()
            pltpu.make_async_copy(bb, gn_hbm.at[pl.ds(base, CH)], sem).wait()

        slots = [(a0, b0, sg0), (a1, b1, sg1), (a2, b2, sg2), (a3, b3, sg3)]
        wsems = [sw0, sw1, sw2, sw3]
        nmain = NCHUNK - 1

        for b in range(4):
            ba, bb, sem = slots[b]
            fire_g(b, ba, bb, sem)

        def body(j, carry):
            for b in range(4):
                k = 4 * j + b
                ba, bb, sem = slots[b]
                wait_g(ba, bb, sem)
                fire_w(k, ba, bb, wsems[b])
            for b in range(4):
                k2 = 4 * j + 4 + b
                ba, bb, sem = slots[b]

                @pl.when(k2 < nmain)
                def _(k2=k2, ba=ba, bb=bb, sem=sem, b=b):
                    wait_w(ba, bb, wsems[b])
                    fire_g(k2, ba, bb, sem)

            return carry

        lax.fori_loop(0, nmain // 4, body, 0)
        for b in range(4):
            ba, bb, _ = slots[b]
            wait_w(ba, bb, wsems[b])

        kt = NCHUNK - 1
        fire_g(kt, a0, b0, sg0)
        wait_g(a0, b0, sg0)
        fire_w(kt, a0, b0, sw0)
        wait_w(a0, b0, sw0)

    return sc_gather


def _sc_gather(zs, zn, sidx, nidx):
    return _build_sc_gather()(zs, zn, sidx, nidx)


@functools.lru_cache(maxsize=None)
def _build_sc_scatter():
    mesh = plsc.VectorSubcoreMesh(core_axis_name="c", subcore_axis_name="s")

    @functools.partial(
        pl.kernel,
        out_type=jax.ShapeDtypeStruct((NC, N, FEA), jnp.float32),
        mesh=mesh,
        scratch_types=[
            pltpu.VMEM((NCHUNK, CH), jnp.int32),
            pltpu.VMEM_SHARED((N, FEA), jnp.float32),
        ] + [pltpu.VMEM((CH, FEA), jnp.float32)] * 3
          + [pltpu.SemaphoreType.DMA] * 6,
    )
    def sc_scatter(msg_hbm, sidx_hbm, zeros_hbm, out_hbm,
                   idx_v, acc_shared, m0, m1, m2,
                   sl0, sl1, sl2, sa0, sa1, sa2):
        c = lax.axis_index("c")
        s = lax.axis_index("s")
        row0 = s * ROWS_PER_SUB
        wid = s * NC + c
        base = wid * EPW
        pltpu.sync_copy(sidx_hbm.at[wid], idx_v)
        pltpu.sync_copy(zeros_hbm.at[pl.ds(row0, ROWS_PER_SUB)],
                        acc_shared.at[pl.ds(row0, ROWS_PER_SUB)])

        @pl.when(s == NS - 1)
        def _():
            pltpu.sync_copy(zeros_hbm.at[pl.ds(TAIL_ROW0, TAIL_ROWS)],
                            acc_shared.at[pl.ds(TAIL_ROW0, TAIL_ROWS)])

        plsc.subcore_barrier()

        slots = [(m0, sl0, sa0), (m1, sl1, sa1), (m2, sl2, sa2)]
        nmain = NCHUNK - 2

        def fire_load(k, buf, sem):
            return pltpu.async_copy(msg_hbm.at[pl.ds(base + k * CH, CH)],
                                    buf, sem)

        def wait_load(buf, sem):
            pltpu.make_async_copy(msg_hbm.at[pl.ds(base, CH)],
                                  buf, sem).wait()

        def fire_sc(k, buf, sem):
            return pltpu.async_copy(buf, acc_shared.at[idx_v.at[k]], sem,
                                    add=True)

        def wait_sc(buf, sem):
            pltpu.make_async_copy(buf, acc_shared.at[idx_v.at[0]],
                                  sem).wait()

        for b in range(3):
            buf, sl, _ = slots[b]
            fire_load(b, buf, sl)

        def body(j, carry):
            for b in range(3):
                k = 3 * j + b
                buf, sl, sa = slots[b]
                wait_load(buf, sl)
                fire_sc(k, buf, sa)
            for b in range(3):
                k2 = 3 * j + 3 + b
                buf, sl, sa = slots[b]

                @pl.when(k2 < nmain)
                def _(k2=k2, buf=buf, sl=sl, sa=sa):
                    wait_sc(buf, sa)
                    fire_load(k2, buf, sl)

            return carry

        lax.fori_loop(0, nmain // 3, body, 0)
        for b in range(3):
            buf, _, sa = slots[b]
            wait_sc(buf, sa)

        for kt in (NCHUNK - 2, NCHUNK - 1):
            lt = fire_load(kt, m0, sl0)
            lt.wait()
            pltpu.sync_copy(m0, acc_shared.at[idx_v.at[kt]], add=True)

        plsc.subcore_barrier()
        pltpu.sync_copy(acc_shared.at[pl.ds(row0, ROWS_PER_SUB)],
                        out_hbm.at[c].at[pl.ds(row0, ROWS_PER_SUB)])

        @pl.when(s == NS - 1)
        def _():
            pltpu.sync_copy(acc_shared.at[pl.ds(TAIL_ROW0, TAIL_ROWS)],
                            out_hbm.at[c].at[pl.ds(TAIL_ROW0, TAIL_ROWS)])

    return sc_scatter


def _sc_scatter(msg, sidx, zeros):
    return _build_sc_scatter()(msg, sidx, zeros)


def kernel(atom_fea, nbr_fea, self_idx, nbr_idx, params):
    p = params
    convs = p['convs']
    sidx = self_idx.reshape(NW, NCHUNK, CH)
    nidx = nbr_idx.reshape(NW, NCHUNK, CH)
    zeros_nf = jnp.zeros((N, FEA), jnp.float32)

    def wsplit(c):
        return (c['fc_W'][:, :FEA].T, c['fc_W'][:, FEA:2 * FEA].T,
                c['fc_W'][:, 2 * FEA:].T, c['fc_b'][None, :])

    wst, wnt, wft, fb = wsplit(convs[0])
    x, zs, zn = _embed_z(atom_fea, p['emb_W'].T, p['emb_b'][None, :],
                         wst, wnt, fb)

    for li, c in enumerate(convs):
        gs, gn = _sc_gather(zs, zn, sidx, nidx)

        bg1 = c['bn1_g'].reshape(2, FEA)
        bb1 = c['bn1_b'].reshape(2, FEA)
        msg = _statsmsg(gs, gn, nbr_fea, wft, bg1, bb1)
        parts = _sc_scatter(msg, sidx, zeros_nf)
        g2 = c['bn2_g'][None, :]
        b2 = c['bn2_b'][None, :]
        if li + 1 < len(convs):
            wst, wnt, wft, fb = wsplit(convs[li + 1])
            x, zs, zn = _final_z(x, parts, g2, b2, wst, wnt, fb)
        else:
            x = _final(x, parts, g2, b2)

    return x


# speedup vs baseline: 2.9050x; 1.1012x over previous
import functools

import jax
import jax.numpy as jnp
from jax import lax
from jax.experimental import pallas as pl
from jax.experimental.pallas import tpu as pltpu
from jax.experimental.pallas import tpu_sc as plsc

N = 10000
E = 320000
FEA = 128
NBR = 16
F2 = 256

NC = 2
NS = 16
NW = NC * NS
EPW = E // NW
CH = 80
NCHUNK = EPW // CH
ROWS_PER_SUB = 624
TAIL_ROW0 = NS * ROWS_PER_SUB
TAIL_ROWS = N - TAIL_ROW0


def _pack16(v):
    b = jax.lax.bitcast_convert_type(v.astype(jnp.bfloat16), jnp.int16)
    return b.astype(jnp.int32) & 0xFFFF


def _store_z(z, z_ref):
    z_ref[...] = _pack16(z[:, :FEA]) | (_pack16(z[:, FEA:]) << 16)


def _unpack16(g):
    y0 = jax.lax.bitcast_convert_type(g << 16, jnp.float32)
    y1 = jax.lax.bitcast_convert_type(g & jnp.int32(-65536), jnp.float32)
    return y0, y1


def _embed_z_body(a_ref, ewt_ref, eb_ref, wst_ref, wnt_ref, fb_ref,
                  x_ref, zs_ref, zn_ref):
    x = jnp.dot(a_ref[...], ewt_ref[...],
                preferred_element_type=jnp.float32) + eb_ref[...]
    x_ref[...] = x
    _store_z(jnp.dot(x, wst_ref[...], preferred_element_type=jnp.float32),
             zs_ref)
    _store_z(jnp.dot(x, wnt_ref[...], preferred_element_type=jnp.float32)
             + fb_ref[...], zn_ref)


def _embed_z(atom_fea, ewt, eb, wst, wnt, fb):
    full = lambda *s: pl.BlockSpec(s, lambda: tuple(0 for _ in s))
    return pl.pallas_call(
        _embed_z_body,
        in_specs=[
            full(N, FEA), full(FEA, FEA), full(1, FEA),
            full(FEA, F2), full(FEA, F2), full(1, F2),
        ],
        out_specs=[full(N, FEA), full(N, FEA), full(N, FEA)],
        out_shape=[
            jax.ShapeDtypeStruct((N, FEA), jnp.float32),
            jax.ShapeDtypeStruct((N, FEA), jnp.int32),
            jax.ShapeDtypeStruct((N, FEA), jnp.int32),
        ],
    )(atom_fea, ewt, eb, wst, wnt, fb)


def _bn2_res(x, parts, g2, b2):
    s = parts[0] + parts[1]
    mu = jnp.mean(s, axis=0, keepdims=True)
    var = jnp.mean(s * s, axis=0, keepdims=True) - mu * mu
    sn = (s - mu) / jnp.sqrt(var + 1e-5) * g2 + b2
    return _softplus(x + sn)


def _final_z_body(x_ref, p_ref, g2_ref, b2_ref, wst_ref, wnt_ref, fb_ref,
                  x_out, zs_ref, zn_ref):
    x = _bn2_res(x_ref[...], p_ref, g2_ref[...], b2_ref[...])
    x_out[...] = x
    _store_z(jnp.dot(x, wst_ref[...], preferred_element_type=jnp.float32),
             zs_ref)
    _store_z(jnp.dot(x, wnt_ref[...], preferred_element_type=jnp.float32)
             + fb_ref[...], zn_ref)


def _final_z(x, parts, g2, b2, wst, wnt, fb):
    full = lambda *s: pl.BlockSpec(s, lambda: tuple(0 for _ in s))
    return pl.pallas_call(
        _final_z_body,
        in_specs=[
            full(N, FEA), full(NC, N, FEA), full(1, FEA), full(1, FEA),
            full(FEA, F2), full(FEA, F2), full(1, F2),
        ],
        out_specs=[full(N, FEA), full(N, FEA), full(N, FEA)],
        out_shape=[
            jax.ShapeDtypeStruct((N, FEA), jnp.float32),
            jax.ShapeDtypeStruct((N, FEA), jnp.int32),
            jax.ShapeDtypeStruct((N, FEA), jnp.int32),
        ],
    )(x, parts, g2, b2, wst, wnt, fb)


def _softplus(u):
    return jnp.maximum(u, 0.0) + jnp.log1p(jnp.exp(-jnp.abs(u)))


def _statsmsg_body(gs_ref, gn_ref, nbr_ref, wft_ref, bg_ref, bb_ref,
                   o_ref, acc):
    p = pl.program_id(0)
    i = pl.program_id(1)
    c = jnp.dot(nbr_ref[...], wft_ref[...],
                preferred_element_type=jnp.float32)
    a0, a1 = _unpack16(gs_ref[...])
    b0, b1 = _unpack16(gn_ref[...])
    y0 = a0 + b0 + c[:, :FEA]
    y1 = a1 + b1 + c[:, FEA:]

    @pl.when((p == 0) & (i == 0))
    def _():
        acc[...] = jnp.zeros_like(acc)

    @pl.when(p == 0)
    def _():
        acc[0:1, :] += jnp.sum(y0, axis=0, keepdims=True)
        acc[1:2, :] += jnp.sum(y1, axis=0, keepdims=True)
        acc[2:3, :] += jnp.sum(y0 * y0, axis=0, keepdims=True)
        acc[3:4, :] += jnp.sum(y1 * y1, axis=0, keepdims=True)

    @pl.when((p == 0) & (i == pl.num_programs(1) - 1))
    def _():
        mu0 = acc[0:1, :] * (1.0 / E)
        mu1 = acc[1:2, :] * (1.0 / E)
        var0 = acc[2:3, :] * (1.0 / E) - mu0 * mu0
        var1 = acc[3:4, :] * (1.0 / E) - mu1 * mu1
        sc0 = bg_ref[0:1, :] * lax.rsqrt(var0 + 1e-5)
        sc1 = bg_ref[1:2, :] * lax.rsqrt(var1 + 1e-5)
        acc[0:1, :] = sc0
        acc[1:2, :] = sc1
        acc[2:3, :] = bb_ref[0:1, :] - mu0 * sc0
        acc[3:4, :] = bb_ref[1:2, :] - mu1 * sc1

    @pl.when(p == 1)
    def _():
        z0 = y0 * acc[0:1, :] + acc[2:3, :]
        z1 = y1 * acc[1:2, :] + acc[3:4, :]
        filt = 1.0 / (1.0 + jnp.exp(-z0))
        core = _softplus(z1)
        o_ref[...] = filt * core


def _statsmsg(gs, gn, nbr_fea, wft, bg, bb):
    te = 4000
    return pl.pallas_call(
        _statsmsg_body,
        grid=(2, E // te),
        in_specs=[
            pl.BlockSpec((te, FEA), lambda p, i: (i, 0)),
            pl.BlockSpec((te, FEA), lambda p, i: (i, 0)),
            pl.BlockSpec((te, NBR), lambda p, i: (i, 0)),
            pl.BlockSpec((NBR, F2), lambda p, i: (0, 0)),
            pl.BlockSpec((2, FEA), lambda p, i: (0, 0)),
            pl.BlockSpec((2, FEA), lambda p, i: (0, 0)),
        ],
        out_specs=pl.BlockSpec((te, FEA), lambda p, i: (p * i, 0)),
        out_shape=jax.ShapeDtypeStruct((E, FEA), jnp.float32),
        scratch_shapes=[pltpu.VMEM((4, FEA), jnp.float32)],
    )(gs, gn, nbr_fea, wft, bg, bb)


def _final_body(x_ref, p_ref, g2_ref, b2_ref, o_ref):
    o_ref[...] = _bn2_res(x_ref[...], p_ref, g2_ref[...], b2_ref[...])


def _final(x, parts, g2, b2):
    full = lambda *s: pl.BlockSpec(s, lambda: tuple(0 for _ in s))
    return pl.pallas_call(
        _final_body,
        in_specs=[
            full(N, FEA), full(NC, N, FEA), full(1, FEA), full(1, FEA),
        ],
        out_specs=full(N, FEA),
        out_shape=jax.ShapeDtypeStruct((N, FEA), jnp.float32),
    )(x, parts, g2, b2)


@functools.lru_cache(maxsize=None)
def _build_sc_gather():
    mesh = plsc.VectorSubcoreMesh(core_axis_name="c", subcore_axis_name="s")

    @functools.partial(
        pl.kernel,
        out_type=[
            jax.ShapeDtypeStruct((E, FEA), jnp.int32),
            jax.ShapeDtypeStruct((E, FEA), jnp.int32),
        ],
        mesh=mesh,
        scratch_types=[
            pltpu.VMEM((NCHUNK, CH), jnp.int32),
            pltpu.VMEM((NCHUNK, CH), jnp.int32),
        ] + [pltpu.VMEM((CH, FEA), jnp.int32)] * 8
          + [pltpu.SemaphoreType.DMA] * 8,
    )
    def sc_gather(zs_hbm, zn_hbm, sidx_hbm, nidx_hbm, gs_hbm, gn_hbm,
                  idx_s, idx_n, a0, b0, a1, b1, a2, b2, a3, b3,
                  sg0, sg1, sg2, sg3, sw0, sw1, sw2, sw3):
        wid = lax.axis_index("s") * NC + lax.axis_index("c")
        base = wid * EPW
        pltpu.sync_copy(sidx_hbm.at[wid], idx_s)
        pltpu.sync_copy(nidx_hbm.at[wid], idx_n)

        def fire_g(k, ba, bb, sem):
            ca = pltpu.async_copy(zs_hbm.at[idx_s.at[k]], ba, sem)
            cb = pltpu.async_copy(zn_hbm.at[idx_n.at[k]], bb, sem)
            return ca, cb

        def wait_g(ba, bb, sem):
            pltpu.make_async_copy(zs_hbm.at[idx_s.at[0]], ba, sem).wait()
            pltpu.make_async_copy(zn_hbm.at[idx_n.at[0]], bb, sem).wait()

        def fire_w(k, ba, bb, sem):
            ca = pltpu.async_copy(ba, gs_hbm.at[pl.ds(base + k * CH, CH)],
                                  sem)
            cb = pltpu.async_copy(bb, gn_hbm.at[pl.ds(base + k * CH, CH)],
                                  sem)
            return ca, cb

        def wait_w(ba, bb, sem):
            pltpu.make_async_copy(ba, gs_hbm.at[pl.ds(base, CH)], sem).wait()
            pltpu.make_async_copy(bb, gn_hbm.at[pl.ds(base, CH)], sem).wait()

        slots = [(a0, b0, sg0), (a1, b1, sg1), (a2, b2, sg2), (a3, b3, sg3)]
        wsems = [sw0, sw1, sw2, sw3]
        nmain = NCHUNK - 1

        for b in range(4):
            ba, bb, sem = slots[b]
            fire_g(b, ba, bb, sem)

        def body(j, carry):
            for b in range(4):
                k = 4 * j + b
                ba, bb, sem = slots[b]
                wait_g(ba, bb, sem)
                fire_w(k, ba, bb, wsems[b])
            for b in range(4):
                k2 = 4 * j + 4 + b
                ba, bb, sem = slots[b]

                @pl.when(k2 < nmain)
                def _(k2=k2, ba=ba, bb=bb, sem=sem, b=b):
                    wait_w(ba, bb, wsems[b])
                    fire_g(k2, ba, bb, sem)

            return carry

        lax.fori_loop(0, nmain // 4, body, 0)
        for b in range(4):
            ba, bb, _ = slots[b]
            wait_w(ba, bb, wsems[b])

        kt = NCHUNK - 1
        fire_g(kt, a0, b0, sg0)
        wait_g(a0, b0, sg0)
        fire_w(kt, a0, b0, sw0)
        wait_w(a0, b0, sw0)

    return sc_gather


def _sc_gather(zs, zn, sidx, nidx):
    return _build_sc_gather()(zs, zn, sidx, nidx)


@functools.lru_cache(maxsize=None)
def _build_sc_scatter():
    mesh = plsc.VectorSubcoreMesh(core_axis_name="c", subcore_axis_name="s")

    @functools.partial(
        pl.kernel,
        out_type=jax.ShapeDtypeStruct((NC, N, FEA), jnp.float32),
        mesh=mesh,
        scratch_types=[
            pltpu.VMEM((NCHUNK, CH), jnp.int32),
            pltpu.VMEM_SHARED((N, FEA), jnp.float32),
        ] + [pltpu.VMEM((CH, FEA), jnp.float32)] * 3
          + [pltpu.SemaphoreType.DMA] * 6,
    )
    def sc_scatter(msg_hbm, sidx_hbm, zeros_hbm, out_hbm,
                   idx_v, acc_shared, m0, m1, m2,
                   sl0, sl1, sl2, sa0, sa1, sa2):
        c = lax.axis_index("c")
        s = lax.axis_index("s")
        row0 = s * ROWS_PER_SUB
        wid = s * NC + c
        base = wid * EPW
        pltpu.sync_copy(sidx_hbm.at[wid], idx_v)
        pltpu.sync_copy(zeros_hbm.at[pl.ds(row0, ROWS_PER_SUB)],
                        acc_shared.at[pl.ds(row0, ROWS_PER_SUB)])

        @pl.when(s == NS - 1)
        def _():
            pltpu.sync_copy(zeros_hbm.at[pl.ds(TAIL_ROW0, TAIL_ROWS)],
                            acc_shared.at[pl.ds(TAIL_ROW0, TAIL_ROWS)])

        plsc.subcore_barrier()

        slots = [(m0, sl0, sa0), (m1, sl1, sa1), (m2, sl2, sa2)]
        nmain = NCHUNK - 2

        def fire_load(k, buf, sem):
            return pltpu.async_copy(msg_hbm.at[pl.ds(base + k * CH, CH)],
                                    buf, sem)

        def wait_load(buf, sem):
            pltpu.make_async_copy(msg_hbm.at[pl.ds(base, CH)],
                                  buf, sem).wait()

        def fire_sc(k, buf, sem):
            return pltpu.async_copy(buf, acc_shared.at[idx_v.at[k]], sem,
                                    add=True)

        def wait_sc(buf, sem):
            pltpu.make_async_copy(buf, acc_shared.at[idx_v.at[0]],
                                  sem).wait()

        for b in range(3):
            buf, sl, _ = slots[b]
            fire_load(b, buf, sl)

        def body(j, carry):
            for b in range(3):
                k = 3 * j + b
                buf, sl, sa = slots[b]
                wait_load(buf, sl)
                fire_sc(k, buf, sa)
            for b in range(3):
                k2 = 3 * j + 3 + b
                buf, sl, sa = slots[b]

                @pl.when(k2 < nmain)
                def _(k2=k2, buf=buf, sl=sl, sa=sa):
                    wait_sc(buf, sa)
                    fire_load(k2, buf, sl)

            return carry

        lax.fori_loop(0, nmain // 3, body, 0)
        for b in range(3):
            buf, _, sa = slots[b]
            wait_sc(buf, sa)

        for kt in (NCHUNK - 2, NCHUNK - 1):
            lt = fire_load(kt, m0, sl0)
            lt.wait()
            pltpu.sync_copy(m0, acc_shared.at[idx_v.at[kt]], add=True)

        plsc.subcore_barrier()
        pltpu.sync_copy(acc_shared.at[pl.ds(row0, ROWS_PER_SUB)],
                        out_hbm.at[c].at[pl.ds(row0, ROWS_PER_SUB)])

        @pl.when(s == NS - 1)
        def _():
            pltpu.sync_copy(acc_shared.at[pl.ds(TAIL_ROW0, TAIL_ROWS)],
                            out_hbm.at[c].at[pl.ds(TAIL_ROW0, TAIL_ROWS)])

    return sc_scatter


def _sc_scatter(msg, sidx, zeros):
    return _build_sc_scatter()(msg, sidx, zeros)


def kernel(atom_fea, nbr_fea, self_idx, nbr_idx, params):
    p = params
    convs = p['convs']
    sidx = self_idx.reshape(NW, NCHUNK, CH)
    nidx = nbr_idx.reshape(NW, NCHUNK, CH)
    zeros_nf = jnp.zeros((N, FEA), jnp.float32)

    def wsplit(c):
        return (c['fc_W'][:, :FEA].T, c['fc_W'][:, FEA:2 * FEA].T,
                c['fc_W'][:, 2 * FEA:].T, c['fc_b'][None, :])

    wst, wnt, wft, fb = wsplit(convs[0])
    x, zs, zn = _embed_z(atom_fea, p['emb_W'].T, p['emb_b'][None, :],
                         wst, wnt, fb)

    for li, c in enumerate(convs):
        gs, gn = _sc_gather(zs, zn, sidx, nidx)

        bg1 = c['bn1_g'].reshape(2, FEA)
        bb1 = c['bn1_b'].reshape(2, FEA)
        msg = _statsmsg(gs, gn, nbr_fea, wft, bg1, bb1)
        parts = _sc_scatter(msg, sidx, zeros_nf)
        g2 = c['bn2_g'][None, :]
        b2 = c['bn2_b'][None, :]
        if li + 1 < len(convs):
            wst, wnt, wft, fb = wsplit(convs[li + 1])
            x, zs, zn = _final_z(x, parts, g2, b2, wst, wnt, fb)
        else:
            x = _final(x, parts, g2, b2)

    return x


# speedup vs baseline: 3.0461x; 1.0486x over previous
import functools

import jax
import jax.numpy as jnp
from jax import lax
from jax.experimental import pallas as pl
from jax.experimental.pallas import tpu as pltpu
from jax.experimental.pallas import tpu_sc as plsc

N = 10000
E = 320000
FEA = 128
NBR = 16
F2 = 256

NC = 2
NS = 16
NW = NC * NS
EPW = E // NW
CH = 80
NCHUNK = EPW // CH
ROWS_PER_SUB = 624
TAIL_ROW0 = NS * ROWS_PER_SUB
TAIL_ROWS = N - TAIL_ROW0


def _pack16(v):
    b = jax.lax.bitcast_convert_type(v.astype(jnp.bfloat16), jnp.int16)
    return b.astype(jnp.int32) & 0xFFFF


def _store_z(z, z_ref):
    z_ref[...] = _pack16(z[:, :FEA]) | (_pack16(z[:, FEA:]) << 16)


def _unpack16(g):
    y0 = jax.lax.bitcast_convert_type(g << 16, jnp.float32)
    y1 = jax.lax.bitcast_convert_type(g & jnp.int32(-65536), jnp.float32)
    return y0, y1


def _embed_z_body(a_ref, ewt_ref, eb_ref, wst_ref, wnt_ref, fb_ref,
                  x_ref, zs_ref, zn_ref):
    x = jnp.dot(a_ref[...], ewt_ref[...],
                preferred_element_type=jnp.float32) + eb_ref[...]
    x_ref[...] = x
    _store_z(jnp.dot(x, wst_ref[...], preferred_element_type=jnp.float32),
             zs_ref)
    _store_z(jnp.dot(x, wnt_ref[...], preferred_element_type=jnp.float32)
             + fb_ref[...], zn_ref)


def _embed_z(atom_fea, ewt, eb, wst, wnt, fb):
    full = lambda *s: pl.BlockSpec(s, lambda: tuple(0 for _ in s))
    return pl.pallas_call(
        _embed_z_body,
        in_specs=[
            full(N, FEA), full(FEA, FEA), full(1, FEA),
            full(FEA, F2), full(FEA, F2), full(1, F2),
        ],
        out_specs=[full(N, FEA), full(N, FEA), full(N, FEA)],
        out_shape=[
            jax.ShapeDtypeStruct((N, FEA), jnp.float32),
            jax.ShapeDtypeStruct((N, FEA), jnp.int32),
            jax.ShapeDtypeStruct((N, FEA), jnp.int32),
        ],
    )(atom_fea, ewt, eb, wst, wnt, fb)


def _bn2_res(x, parts, g2, b2):
    s = parts[0] + parts[1]
    mu = jnp.mean(s, axis=0, keepdims=True)
    var = jnp.mean(s * s, axis=0, keepdims=True) - mu * mu
    sn = (s - mu) / jnp.sqrt(var + 1e-5) * g2 + b2
    return _softplus(x + sn)


def _final_z_body(x_ref, p_ref, g2_ref, b2_ref, wst_ref, wnt_ref, fb_ref,
                  x_out, zs_ref, zn_ref):
    x = _bn2_res(x_ref[...], p_ref, g2_ref[...], b2_ref[...])
    x_out[...] = x
    _store_z(jnp.dot(x, wst_ref[...], preferred_element_type=jnp.float32),
             zs_ref)
    _store_z(jnp.dot(x, wnt_ref[...], preferred_element_type=jnp.float32)
             + fb_ref[...], zn_ref)


def _final_z(x, parts, g2, b2, wst, wnt, fb):
    full = lambda *s: pl.BlockSpec(s, lambda: tuple(0 for _ in s))
    return pl.pallas_call(
        _final_z_body,
        in_specs=[
            full(N, FEA), full(NC, N, FEA), full(1, FEA), full(1, FEA),
            full(FEA, F2), full(FEA, F2), full(1, F2),
        ],
        out_specs=[full(N, FEA), full(N, FEA), full(N, FEA)],
        out_shape=[
            jax.ShapeDtypeStruct((N, FEA), jnp.float32),
            jax.ShapeDtypeStruct((N, FEA), jnp.int32),
            jax.ShapeDtypeStruct((N, FEA), jnp.int32),
        ],
    )(x, parts, g2, b2, wst, wnt, fb)


def _softplus(u):
    return jnp.maximum(u, 0.0) + jnp.log1p(jnp.exp(-jnp.abs(u)))


def _statsmsg_body(gs_ref, gn_ref, nbr_ref, wft_ref, bg_ref, bb_ref,
                   o_ref, acc):
    p = pl.program_id(0)
    i = pl.program_id(1)
    c = jnp.dot(nbr_ref[...], wft_ref[...],
                preferred_element_type=jnp.float32)
    a0, a1 = _unpack16(gs_ref[...])
    b0, b1 = _unpack16(gn_ref[...])
    y0 = a0 + b0 + c[:, :FEA]
    y1 = a1 + b1 + c[:, FEA:]

    @pl.when((p == 0) & (i == 0))
    def _():
        acc[...] = jnp.zeros_like(acc)

    @pl.when(p == 0)
    def _():
        acc[0:1, :] += jnp.sum(y0, axis=0, keepdims=True)
        acc[1:2, :] += jnp.sum(y1, axis=0, keepdims=True)
        acc[2:3, :] += jnp.sum(y0 * y0, axis=0, keepdims=True)
        acc[3:4, :] += jnp.sum(y1 * y1, axis=0, keepdims=True)

    @pl.when((p == 0) & (i == pl.num_programs(1) - 1))
    def _():
        mu0 = acc[0:1, :] * (1.0 / E)
        mu1 = acc[1:2, :] * (1.0 / E)
        var0 = acc[2:3, :] * (1.0 / E) - mu0 * mu0
        var1 = acc[3:4, :] * (1.0 / E) - mu1 * mu1
        sc0 = bg_ref[0:1, :] * lax.rsqrt(var0 + 1e-5)
        sc1 = bg_ref[1:2, :] * lax.rsqrt(var1 + 1e-5)
        acc[0:1, :] = sc0
        acc[1:2, :] = sc1
        acc[2:3, :] = bb_ref[0:1, :] - mu0 * sc0
        acc[3:4, :] = bb_ref[1:2, :] - mu1 * sc1

    @pl.when(p == 1)
    def _():
        z0 = y0 * acc[0:1, :] + acc[2:3, :]
        z1 = y1 * acc[1:2, :] + acc[3:4, :]
        filt = 1.0 / (1.0 + jnp.exp(-z0))
        core = _softplus(z1)
        o_ref[...] = filt * core


def _statsmsg(gs, gn, nbr_fea, wft, bg, bb):
    te = 8000
    return pl.pallas_call(
        _statsmsg_body,
        grid=(2, E // te),
        in_specs=[
            pl.BlockSpec((te, FEA), lambda p, i: (i, 0)),
            pl.BlockSpec((te, FEA), lambda p, i: (i, 0)),
            pl.BlockSpec((te, NBR), lambda p, i: (i, 0)),
            pl.BlockSpec((NBR, F2), lambda p, i: (0, 0)),
            pl.BlockSpec((2, FEA), lambda p, i: (0, 0)),
            pl.BlockSpec((2, FEA), lambda p, i: (0, 0)),
        ],
        out_specs=pl.BlockSpec((te, FEA), lambda p, i: (p * i, 0)),
        out_shape=jax.ShapeDtypeStruct((E, FEA), jnp.float32),
        scratch_shapes=[pltpu.VMEM((4, FEA), jnp.float32)],
    )(gs, gn, nbr_fea, wft, bg, bb)


def _final_body(x_ref, p_ref, g2_ref, b2_ref, o_ref):
    o_ref[...] = _bn2_res(x_ref[...], p_ref, g2_ref[...], b2_ref[...])


def _final(x, parts, g2, b2):
    full = lambda *s: pl.BlockSpec(s, lambda: tuple(0 for _ in s))
    return pl.pallas_call(
        _final_body,
        in_specs=[
            full(N, FEA), full(NC, N, FEA), full(1, FEA), full(1, FEA),
        ],
        out_specs=full(N, FEA),
        out_shape=jax.ShapeDtypeStruct((N, FEA), jnp.float32),
    )(x, parts, g2, b2)


@functools.lru_cache(maxsize=None)
def _build_sc_gather():
    mesh = plsc.VectorSubcoreMesh(core_axis_name="c", subcore_axis_name="s")

    @functools.partial(
        pl.kernel,
        out_type=[
            jax.ShapeDtypeStruct((E, FEA), jnp.int32),
            jax.ShapeDtypeStruct((E, FEA), jnp.int32),
        ],
        mesh=mesh,
        scratch_types=[
            pltpu.VMEM((NCHUNK, CH), jnp.int32),
            pltpu.VMEM((NCHUNK, CH), jnp.int32),
        ] + [pltpu.VMEM((CH, FEA), jnp.int32)] * 8
          + [pltpu.SemaphoreType.DMA] * 8,
    )
    def sc_gather(zs_hbm, zn_hbm, sidx_hbm, nidx_hbm, gs_hbm, gn_hbm,
                  idx_s, idx_n, a0, b0, a1, b1, a2, b2, a3, b3,
                  sg0, sg1, sg2, sg3, sw0, sw1, sw2, sw3):
        wid = lax.axis_index("s") * NC + lax.axis_index("c")
        base = wid * EPW
        pltpu.sync_copy(sidx_hbm.at[wid], idx_s)
        pltpu.sync_copy(nidx_hbm.at[wid], idx_n)

        def fire_g(k, ba, bb, sem):
            ca = pltpu.async_copy(zs_hbm.at[idx_s.at[k]], ba, sem)
            cb = pltpu.async_copy(zn_hbm.at[idx_n.at[k]], bb, sem)
            return ca, cb

        def wait_g(ba, bb, sem):
            pltpu.make_async_copy(zs_hbm.at[idx_s.at[0]], ba, sem).wait()
            pltpu.make_async_copy(zn_hbm.at[idx_n.at[0]], bb, sem).wait()

        def fire_w(k, ba, bb, sem):
            ca = pltpu.async_copy(ba, gs_hbm.at[pl.ds(base + k * CH, CH)],
                                  sem)
            cb = pltpu.async_copy(bb, gn_hbm.at[pl.ds(base + k * CH, CH)],
                                  sem)
            return ca, cb

        def wait_w(ba, bb, sem):
            pltpu.make_async_copy(ba, gs_hbm.at[pl.ds(base, CH)], sem).wait()
            pltpu.make_async_copy(bb, gn_hbm.at[pl.ds(base, CH)], sem).wait()

        slots = [(a0, b0, sg0), (a1, b1, sg1), (a2, b2, sg2), (a3, b3, sg3)]
        wsems = [sw0, sw1, sw2, sw3]
        nmain = NCHUNK - 1

        for b in range(4):
            ba, bb, sem = slots[b]
            fire_g(b, ba, bb, sem)

        def body(j, carry):
            for b in range(4):
                k = 4 * j + b
                ba, bb, sem = slots[b]
                wait_g(ba, bb, sem)
                fire_w(k, ba, bb, wsems[b])
            for b in range(4):
                k2 = 4 * j + 4 + b
                ba, bb, sem = slots[b]

                @pl.when(k2 < nmain)
                def _(k2=k2, ba=ba, bb=bb, sem=sem, b=b):
                    wait_w(ba, bb, wsems[b])
                    fire_g(k2, ba, bb, sem)

            return carry

        lax.fori_loop(0, nmain // 4, body, 0)
        for b in range(4):
            ba, bb, _ = slots[b]
            wait_w(ba, bb, wsems[b])

        kt = NCHUNK - 1
        fire_g(kt, a0, b0, sg0)
        wait_g(a0, b0, sg0)
        fire_w(kt, a0, b0, sw0)
        wait_w(a0, b0, sw0)

    return sc_gather


def _sc_gather(zs, zn, sidx, nidx):
    return _build_sc_gather()(zs, zn, sidx, nidx)


@functools.lru_cache(maxsize=None)
def _build_sc_scatter():
    mesh = plsc.VectorSubcoreMesh(core_axis_name="c", subcore_axis_name="s")

    @functools.partial(
        pl.kernel,
        out_type=jax.ShapeDtypeStruct((NC, N, FEA), jnp.float32),
        mesh=mesh,
        scratch_types=[
            pltpu.VMEM((NCHUNK, CH), jnp.int32),
            pltpu.VMEM_SHARED((N, FEA), jnp.float32),
        ] + [pltpu.VMEM((CH, FEA), jnp.float32)] * 3
          + [pltpu.SemaphoreType.DMA] * 6,
    )
    def sc_scatter(msg_hbm, sidx_hbm, zeros_hbm, out_hbm,
                   idx_v, acc_shared, m0, m1, m2,
                   sl0, sl1, sl2, sa0, sa1, sa2):
        c = lax.axis_index("c")
        s = lax.axis_index("s")
        row0 = s * ROWS_PER_SUB
        wid = s * NC + c
        base = wid * EPW
        pltpu.sync_copy(sidx_hbm.at[wid], idx_v)
        pltpu.sync_copy(zeros_hbm.at[pl.ds(row0, ROWS_PER_SUB)],
                        acc_shared.at[pl.ds(row0, ROWS_PER_SUB)])

        @pl.when(s == NS - 1)
        def _():
            pltpu.sync_copy(zeros_hbm.at[pl.ds(TAIL_ROW0, TAIL_ROWS)],
                            acc_shared.at[pl.ds(TAIL_ROW0, TAIL_ROWS)])

        plsc.subcore_barrier()

        slots = [(m0, sl0, sa0), (m1, sl1, sa1), (m2, sl2, sa2)]
        nmain = NCHUNK - 2

        def fire_load(k, buf, sem):
            return pltpu.async_copy(msg_hbm.at[pl.ds(base + k * CH, CH)],
                                    buf, sem)

        def wait_load(buf, sem):
            pltpu.make_async_copy(msg_hbm.at[pl.ds(base, CH)],
                                  buf, sem).wait()

        def fire_sc(k, buf, sem):
            return pltpu.async_copy(buf, acc_shared.at[idx_v.at[k]], sem,
                                    add=True)

        def wait_sc(buf, sem):
            pltpu.make_async_copy(buf, acc_shared.at[idx_v.at[0]],
                                  sem).wait()

        for b in range(3):
            buf, sl, _ = slots[b]
            fire_load(b, buf, sl)

        def body(j, carry):
            for b in range(3):
                k = 3 * j + b
                buf, sl, sa = slots[b]
                wait_load(buf, sl)
                fire_sc(k, buf, sa)
            for b in range(3):
                k2 = 3 * j + 3 + b
                buf, sl, sa = slots[b]

                @pl.when(k2 < nmain)
                def _(k2=k2, buf=buf, sl=sl, sa=sa):
                    wait_sc(buf, sa)
                    fire_load(k2, buf, sl)

            return carry

        lax.fori_loop(0, nmain // 3, body, 0)
        for b in range(3):
            buf, _, sa = slots[b]
            wait_sc(buf, sa)

        for kt in (NCHUNK - 2, NCHUNK - 1):
            lt = fire_load(kt, m0, sl0)
            lt.wait()
            pltpu.sync_copy(m0, acc_shared.at[idx_v.at[kt]], add=True)

        plsc.subcore_barrier()
        pltpu.sync_copy(acc_shared.at[pl.ds(row0, ROWS_PER_SUB)],
                        out_hbm.at[c].at[pl.ds(row0, ROWS_PER_SUB)])

        @pl.when(s == NS - 1)
        def _():
            pltpu.sync_copy(acc_shared.at[pl.ds(TAIL_ROW0, TAIL_ROWS)],
                            out_hbm.at[c].at[pl.ds(TAIL_ROW0, TAIL_ROWS)])

    return sc_scatter


def _sc_scatter(msg, sidx, zeros):
    return _build_sc_scatter()(msg, sidx, zeros)


def kernel(atom_fea, nbr_fea, self_idx, nbr_idx, params):
    p = params
    convs = p['convs']
    sidx = self_idx.reshape(NW, NCHUNK, CH)
    nidx = nbr_idx.reshape(NW, NCHUNK, CH)
    zeros_nf = jnp.zeros((N, FEA), jnp.float32)

    def wsplit(c):
        return (c['fc_W'][:, :FEA].T, c['fc_W'][:, FEA:2 * FEA].T,
                c['fc_W'][:, 2 * FEA:].T, c['fc_b'][None, :])

    wst, wnt, wft, fb = wsplit(convs[0])
    x, zs, zn = _embed_z(atom_fea, p['emb_W'].T, p['emb_b'][None, :],
                         wst, wnt, fb)

    for li, c in enumerate(convs):
        gs, gn = _sc_gather(zs, zn, sidx, nidx)

        bg1 = c['bn1_g'].reshape(2, FEA)
        bb1 = c['bn1_b'].reshape(2, FEA)
        msg = _statsmsg(gs, gn, nbr_fea, wft, bg1, bb1)
        parts = _sc_scatter(msg, sidx, zeros_nf)
        g2 = c['bn2_g'][None, :]
        b2 = c['bn2_b'][None, :]
        if li + 1 < len(convs):
            wst, wnt, wft, fb = wsplit(convs[li + 1])
            x, zs, zn = _final_z(x, parts, g2, b2, wst, wnt, fb)
        else:
            x = _final(x, parts, g2, b2)

    return x
